# Initial kernel scaffold; baseline (speedup 1.0000x reference)
import jax
import jax.numpy as jnp
from jax.experimental import pallas as pl


def kernel(x, edge_index, batch, embed, vp_w1, vp_b1, vp_w2, vp_b2, vp_ln_g, vp_ln_b, w0, a_src0, a_dst0, bias0, n0_g, n0_b, w1, a_src1, a_dst1, bias1, n1_g, n1_b, ro_w, ro_b):
    raise NotImplementedError("write your pallas kernel here")


# speedup vs baseline: 6.2652x; 6.2652x over previous
import functools

import jax
import jax.numpy as jnp
from jax import lax
from jax.experimental import pallas as pl
from jax.experimental.pallas import tpu as pltpu
from jax.experimental.pallas import tpu_sc as plsc

N = 10000
E = 160000
G = 64
HID = 256

NTILE = 16
EPT = E // NTILE
C = 80
NCHUNK = EPT // C
WB_TILES = 10
WB_ROWS = N // WB_TILES
WB_CH = 40
DEN_W = 16


CE = 2000


def _edge_ex_call(hsc, src, dst, s2_flat):
    mesh = plsc.VectorSubcoreMesh(core_axis_name="c", subcore_axis_name="s")

    def body(src_hbm, dst_hbm, s2_hbm, ex_hbm, s_tab, exb, src_c, dst_c):
        c = lax.axis_index("c")
        t = lax.axis_index("s")
        cN = c * N
        iota16 = jnp.arange(16, dtype=jnp.int32)
        zero16 = jnp.zeros((16,), jnp.float32)

        def zrow(r, _):
            exb[r, pl.ds(0, 16)] = zero16
            return 0
        lax.fori_loop(0, CE, zrow, 0)
        pltpu.sync_copy(s2_hbm.at[pl.ds(cN, N), :], s_tab)

        def chunk(j, _):
            eb = t * EPT + j * CE
            pltpu.sync_copy(src_hbm.at[pl.ds(eb, CE)], src_c)
            pltpu.sync_copy(dst_hbm.at[pl.ds(eb, CE)], dst_c)

            def group(g, _):
                ev = g * 16 + iota16
                sv = src_c[pl.ds(g * 16, 16)]
                dv = dst_c[pl.ds(g * 16, 16)]
                for h in range(hsc):
                    hcol = jnp.full((16,), h, jnp.int32)
                    a = (plsc.load_gather(s_tab, [sv, hcol])
                         + plsc.load_gather(s_tab, [dv, hcol + hsc]))
                    a = jnp.where(a > 0, a, 0.2 * a)
                    plsc.store_scatter(exb, [ev, hcol], jnp.exp(a))
                return 0
            lax.fori_loop(0, CE // 16, group, 0)
            pltpu.sync_copy(exb, ex_hbm.at[pl.ds(c * E + eb, CE), :])
            return 0
        lax.fori_loop(0, EPT // CE, chunk, 0)

    f = pl.kernel(
        body,
        out_type=jax.ShapeDtypeStruct((2 * E, DEN_W), jnp.float32),
        mesh=mesh,
        compiler_params=pltpu.CompilerParams(needs_layout_passes=False, use_tc_tiling_on_sc=False),
        scratch_types=[
            pltpu.VMEM((N, 2 * hsc), jnp.float32),
            pltpu.VMEM((CE, DEN_W), jnp.float32),
            pltpu.VMEM((CE,), jnp.int32),
            pltpu.VMEM((CE,), jnp.int32),
        ],
    )
    return f(src, dst, s2_flat)


def _edge_agg_call(hsc, colw, src, dst, hw_flat, ex_flat):
    mesh = plsc.VectorSubcoreMesh(core_axis_name="c", subcore_axis_name="s")

    def body(src_hbm, dst_hbm, hw_hbm, ex_hbm, acc_hbm, den_hbm,
             acc_sh, den_sh, rows, exb, src_c, dst_c, sem):
        c = lax.axis_index("c")
        t = lax.axis_index("s")
        cN = c * N
        base = t * WB_ROWS
        iota16 = jnp.arange(16, dtype=jnp.int32)
        zero16 = jnp.zeros((16,), jnp.float32)

        def zrow(r, _):
            for v in range(128 // 16):
                rows[r, pl.ds(v * 16, 16)] = zero16
            exb[r, pl.ds(0, 16)] = zero16
            return 0
        lax.fori_loop(0, C, zrow, 0)

        @pl.when(t < WB_TILES)
        def _zero():
            for j in range(WB_ROWS // WB_CH):
                pltpu.sync_copy(rows.at[pl.ds(0, WB_CH), :],
                                acc_sh.at[pl.ds(base + j * WB_CH, WB_CH), :])
                pltpu.sync_copy(exb.at[pl.ds(0, WB_CH), :],
                                den_sh.at[pl.ds(base + j * WB_CH, WB_CH), :])
        plsc.subcore_barrier()

        def chunk(j, _):
            eb = t * EPT + j * C
            pltpu.sync_copy(src_hbm.at[pl.ds(eb, C)], src_c)
            pltpu.sync_copy(dst_hbm.at[pl.ds(eb, C)], dst_c)
            pltpu.sync_copy(ex_hbm.at[pl.ds(c * E + eb, C), :], exb)

            def adj(k, _):
                src_c[pl.ds(k * 16, 16)] = src_c[pl.ds(k * 16, 16)] + cN
                return 0
            lax.fori_loop(0, C // 16, adj, 0)
            pltpu.async_copy(hw_hbm.at[src_c], rows, sem).wait()

            def group(g, _):
                ev = g * 16 + iota16
                for h in range(hsc):
                    ex = plsc.load_gather(
                        exb, [ev, jnp.full((16,), h, jnp.int32)])
                    for col in range(h * colw, (h + 1) * colw):
                        cv = jnp.full((16,), col, jnp.int32)
                        r = plsc.load_gather(rows, [ev, cv])
                        plsc.store_scatter(rows, [ev, cv], r * ex)
                return 0
            lax.fori_loop(0, C // 16, group, 0)

            pltpu.sync_copy(rows, acc_sh.at[dst_c], add=True)
            pltpu.sync_copy(exb, den_sh.at[dst_c], add=True)
            return 0
        lax.fori_loop(0, NCHUNK, chunk, 0)

        plsc.subcore_barrier()

        @pl.when(t < WB_TILES)
        def _writeback():
            for j in range(WB_ROWS // WB_CH):
                o = base + j * WB_CH
                pltpu.sync_copy(acc_sh.at[pl.ds(o, WB_CH), :],
                                acc_hbm.at[pl.ds(cN + o, WB_CH), :])
                pltpu.sync_copy(den_sh.at[pl.ds(o, WB_CH), :],
                                den_hbm.at[pl.ds(cN + o, WB_CH), :])

    f = pl.kernel(
        body,
        out_type=(jax.ShapeDtypeStruct((2 * N, 128), jnp.float32),
                  jax.ShapeDtypeStruct((2 * N, DEN_W), jnp.float32)),
        mesh=mesh,
        compiler_params=pltpu.CompilerParams(needs_layout_passes=False, use_tc_tiling_on_sc=False),
        scratch_types=[
            pltpu.VMEM_SHARED((N, 128), jnp.float32),
            pltpu.VMEM_SHARED((N, DEN_W), jnp.float32),
            pltpu.VMEM((C, 128), jnp.float32),
            pltpu.VMEM((C, DEN_W), jnp.float32),
            pltpu.VMEM((C,), jnp.int32),
            pltpu.VMEM((C,), jnp.int32),
            pltpu.SemaphoreType.DMA,
        ],
    )
    return f(src, dst, hw_flat, ex_flat)


def _edge_sc_call(hsc, colw, src, dst, hw_flat, s2_flat):
    ex_flat = _edge_ex_call(hsc, src, dst, s2_flat)
    return _edge_agg_call(hsc, colw, src, dst, hw_flat, ex_flat)


def _ln(x, g, b, eps=1e-5):
    m = jnp.mean(x, axis=-1, keepdims=True)
    v = jnp.mean((x - m) ** 2, axis=-1, keepdims=True)
    return (x - m) / jnp.sqrt(v + eps) * g + b


def _front_body(vis, ohc, emb, w1, b1, w2, b2, g, b, out):
    h1 = jnp.maximum(jnp.dot(vis[...], w1[...],
                             preferred_element_type=jnp.float32) + b1[...], 0.0)
    v = jnp.dot(h1, w2[...], preferred_element_type=jnp.float32) + b2[...]
    out[:, 0:128] = jnp.dot(ohc[...], emb[...],
                            preferred_element_type=jnp.float32)
    out[:, 128:256] = _ln(v, g[...], b[...])


def _front(vis, ohc, emb, w1, b1, w2, b2, g, b):
    bn = 1000
    grid = (N // bn,)
    return pl.pallas_call(
        _front_body,
        grid=grid,
        in_specs=[
            pl.BlockSpec((bn, 2048), lambda i: (i, 0)),
            pl.BlockSpec((bn, 128), lambda i: (i, 0)),
            pl.BlockSpec((128, 128), lambda i: (0, 0)),
            pl.BlockSpec((2048, 512), lambda i: (0, 0)),
            pl.BlockSpec((1, 512), lambda i: (0, 0)),
            pl.BlockSpec((512, 128), lambda i: (0, 0)),
            pl.BlockSpec((1, 128), lambda i: (0, 0)),
            pl.BlockSpec((1, 128), lambda i: (0, 0)),
            pl.BlockSpec((1, 128), lambda i: (0, 0)),
        ],
        out_specs=pl.BlockSpec((bn, 256), lambda i: (i, 0)),
        out_shape=jax.ShapeDtypeStruct((N, 256), jnp.float32),
    )(vis, ohc, emb, w1, b1, w2, b2, g, b)


def _pre_body(hsc, h, w, asrc, adst, hw2, s2, exs):
    hw = jnp.dot(h[...], w[...], preferred_element_type=jnp.float32)
    ss = jnp.dot(hw, asrc[...], preferred_element_type=jnp.float32)
    sd = jnp.dot(hw, adst[...], preferred_element_type=jnp.float32)
    hw2[0] = hw[:, 0:128]
    hw2[1] = hw[:, 128:256]
    if hsc * 2 == ss.shape[1]:
        s2[0, :, 0:hsc] = ss[:, 0:hsc]
        s2[0, :, hsc:2 * hsc] = sd[:, 0:hsc]
        s2[1, :, 0:hsc] = ss[:, hsc:2 * hsc]
        s2[1, :, hsc:2 * hsc] = sd[:, hsc:2 * hsc]
    else:
        s2[0, :, 0:1] = ss
        s2[0, :, 1:2] = sd
        s2[1, :, 0:1] = ss
        s2[1, :, 1:2] = sd
    a = ss + sd
    a = jnp.where(a > 0, a, 0.2 * a)
    exs[...] = jnp.exp(a)


def _pre(h, w, asrc, adst, heads, hsc):
    bn = 1000
    grid = (N // bn,)
    return pl.pallas_call(
        functools.partial(_pre_body, hsc),
        grid=grid,
        in_specs=[
            pl.BlockSpec((bn, 256), lambda i: (i, 0)),
            pl.BlockSpec((256, 256), lambda i: (0, 0)),
            pl.BlockSpec((256, heads), lambda i: (0, 0)),
            pl.BlockSpec((256, heads), lambda i: (0, 0)),
        ],
        out_specs=[
            pl.BlockSpec((2, bn, 128), lambda i: (0, i, 0)),
            pl.BlockSpec((2, bn, 2 * hsc), lambda i: (0, i, 0)),
            pl.BlockSpec((bn, heads), lambda i: (i, 0)),
        ],
        out_shape=[
            jax.ShapeDtypeStruct((2, N, 128), jnp.float32),
            jax.ShapeDtypeStruct((2, N, 2 * hsc), jnp.float32),
            jax.ShapeDtypeStruct((N, heads), jnp.float32),
        ],
    )(h, w, asrc, adst)


def _post_body(hsc, acc, den, exs, hw2, h, rep, bias, g, b, out):
    num = jnp.concatenate([acc[0], acc[1]], axis=1)
    hwc = jnp.concatenate([hw2[0], hw2[1]], axis=1)
    e = exs[...]
    num = num + jnp.dot(e, rep[...],
                        preferred_element_type=jnp.float32) * hwc
    if hsc * 2 == e.shape[1]:
        denh = jnp.concatenate([den[0][:, 0:hsc], den[1][:, 0:hsc]], axis=1)
    else:
        denh = den[0][:, 0:1]
    d = jnp.dot(denh + e, rep[...], preferred_element_type=jnp.float32)
    xn = num / (d + 1e-16) + bias[...]
    xn = jnp.where(xn > 0, xn, jnp.exp(xn) - 1.0)
    out[...] = _ln(xn + h[...], g[...], b[...])


def _post(acc, den, exs, hw2, h, rep, bias, g, b, heads, hsc):
    bn = 1000
    grid = (N // bn,)
    return pl.pallas_call(
        functools.partial(_post_body, hsc),
        grid=grid,
        in_specs=[
            pl.BlockSpec((2, bn, 128), lambda i: (0, i, 0)),
            pl.BlockSpec((2, bn, DEN_W), lambda i: (0, i, 0)),
            pl.BlockSpec((bn, heads), lambda i: (i, 0)),
            pl.BlockSpec((2, bn, 128), lambda i: (0, i, 0)),
            pl.BlockSpec((bn, 256), lambda i: (i, 0)),
            pl.BlockSpec((heads, 256), lambda i: (0, 0)),
            pl.BlockSpec((1, 256), lambda i: (0, 0)),
            pl.BlockSpec((1, 256), lambda i: (0, 0)),
            pl.BlockSpec((1, 256), lambda i: (0, 0)),
        ],
        out_specs=pl.BlockSpec((bn, 256), lambda i: (i, 0)),
        out_shape=jax.ShapeDtypeStruct((N, 256), jnp.float32),
    )(acc, den, exs, hw2, h, rep, bias, g, b)


def _pool_body(nsteps, h2, oh, row, rob, out, psum, cnt):
    i = pl.program_id(0)

    @pl.when(i == 0)
    def _init():
        psum[...] = jnp.zeros_like(psum)
        cnt[...] = jnp.zeros_like(cnt)

    ohb = oh[...]
    psum[...] += lax.dot_general(ohb, h2[...], (((0,), (0,)), ((), ())),
                                 preferred_element_type=jnp.float32)
    cnt[...] += jnp.sum(ohb, axis=0, keepdims=True)

    @pl.when(i == nsteps - 1)
    def _fin():
        pooled = psum[...] / jnp.maximum(cnt[...], 1.0).reshape(G, 1)
        logit = jnp.dot(pooled, row[...],
                        preferred_element_type=jnp.float32) + rob[...]
        out[...] = 1.0 / (1.0 + jnp.exp(-logit))


def _pool(h2, oh, row, rob):
    bn = 1000
    nsteps = N // bn
    return pl.pallas_call(
        functools.partial(_pool_body, nsteps),
        grid=(nsteps,),
        in_specs=[
            pl.BlockSpec((bn, 256), lambda i: (i, 0)),
            pl.BlockSpec((bn, G), lambda i: (i, 0)),
            pl.BlockSpec((256, 1), lambda i: (0, 0)),
            pl.BlockSpec((1, 1), lambda i: (0, 0)),
        ],
        out_specs=pl.BlockSpec((G, 1), lambda i: (0, 0)),
        out_shape=jax.ShapeDtypeStruct((G, 1), jnp.float32),
        scratch_shapes=[
            pltpu.VMEM((G, 256), jnp.float32),
            pltpu.VMEM((1, G), jnp.float32),
        ],
    )(h2, oh, row, rob)


def _expander(a, heads, oc):
    rows = jnp.repeat(jnp.arange(heads), oc)
    mask = (rows[:, None] == jnp.arange(heads)[None, :]).astype(jnp.float32)
    return a.reshape(heads * oc, 1) * mask


def _rep(heads, colw):
    cols = jnp.arange(256) // colw
    return (jnp.arange(heads)[:, None] == cols[None, :]).astype(jnp.float32)


def _gat_layer(h, w, a_src, a_dst, bias, g, b, src, dst, heads):
    oc = HID // heads
    hsc = max(heads // 2, 1)
    colw = 128 // hsc
    asrc = _expander(a_src, heads, oc)
    adst = _expander(a_dst, heads, oc)
    hw2, s2, exs = _pre(h, w, asrc, adst, heads, hsc)
    acc, den = _edge_sc_call(hsc, colw, src, dst,
                             hw2.reshape(2 * N, 128),
                             s2.reshape(2 * N, 2 * hsc))
    rep = _rep(heads, HID // heads)
    return _post(acc.reshape(2, N, 128), den.reshape(2, N, DEN_W), exs, hw2,
                 h, rep, bias.reshape(1, 256), g.reshape(1, 256),
                 b.reshape(1, 256), heads, hsc)


def kernel(x, edge_index, batch, embed, vp_w1, vp_b1, vp_w2, vp_b2, vp_ln_g,
           vp_ln_b, w0, a_src0, a_dst0, bias0, n0_g, n0_b, w1, a_src1,
           a_dst1, bias1, n1_g, n1_b, ro_w, ro_b):
    vis = x[:, 1:]
    cat = x[:, 0:1].astype(jnp.int32)
    ohc = (cat == jnp.arange(128, dtype=jnp.int32)[None, :]).astype(
        jnp.float32)
    emb = jnp.pad(embed, ((0, 128 - embed.shape[0]), (0, 0)))
    h = _front(vis, ohc, emb, vp_w1, vp_b1.reshape(1, 512), vp_w2,
               vp_b2.reshape(1, 128), vp_ln_g.reshape(1, 128),
               vp_ln_b.reshape(1, 128))

    src = edge_index[0]
    dst = edge_index[1]
    h1 = _gat_layer(h, w0, a_src0, a_dst0, bias0, n0_g, n0_b, src, dst, 4)
    h2 = _gat_layer(h1, w1, a_src1, a_dst1, bias1, n1_g, n1_b, src, dst, 1)

    oh = (batch[:, None] == jnp.arange(G, dtype=batch.dtype)[None, :]).astype(
        jnp.float32)
    score = _pool(h2, oh, ro_w, ro_b.reshape(1, 1))
    return score.reshape(G)


# speedup vs baseline: 7.2375x; 1.1552x over previous
import functools

import jax
import jax.numpy as jnp
from jax import lax
from jax.experimental import pallas as pl
from jax.experimental.pallas import tpu as pltpu
from jax.experimental.pallas import tpu_sc as plsc

N = 10000
E = 160000
G = 64
HID = 256

NTILE = 16
EPT = E // NTILE
C = 80
BT = 2000
NCHUNK = EPT // C
WB_TILES = 10
WB_ROWS = N // WB_TILES
WB_CH = 40
DEN_W = 16


CE = 2000


def _edge_ex_call(hsc, src, dst, s2_flat):
    mesh = plsc.VectorSubcoreMesh(core_axis_name="c", subcore_axis_name="s")

    def body(src_hbm, dst_hbm, s2_hbm, ex_hbm, s_tab, exb, src_c, dst_c):
        c = lax.axis_index("c")
        t = lax.axis_index("s")
        cN = c * N
        iota16 = jnp.arange(16, dtype=jnp.int32)
        zero16 = jnp.zeros((16,), jnp.float32)

        def zrow(r, _):
            exb[r, pl.ds(0, 16)] = zero16
            return 0
        lax.fori_loop(0, CE, zrow, 0)
        pltpu.sync_copy(s2_hbm.at[pl.ds(cN, N), :], s_tab)

        def chunk(j, _):
            eb = t * EPT + j * CE
            pltpu.sync_copy(src_hbm.at[pl.ds(eb, CE)], src_c)
            pltpu.sync_copy(dst_hbm.at[pl.ds(eb, CE)], dst_c)

            def group(g, _):
                ev = g * 16 + iota16
                sv = src_c[pl.ds(g * 16, 16)]
                dv = dst_c[pl.ds(g * 16, 16)]
                for h in range(hsc):
                    hcol = jnp.full((16,), h, jnp.int32)
                    a = (plsc.load_gather(s_tab, [sv, hcol])
                         + plsc.load_gather(s_tab, [dv, hcol + hsc]))
                    a = jnp.where(a > 0, a, 0.2 * a)
                    plsc.store_scatter(exb, [ev, hcol], jnp.exp(a))
                return 0
            lax.fori_loop(0, CE // 16, group, 0)
            pltpu.sync_copy(exb, ex_hbm.at[pl.ds(c * E + eb, CE), :])
            return 0
        lax.fori_loop(0, EPT // CE, chunk, 0)

    f = pl.kernel(
        body,
        out_type=jax.ShapeDtypeStruct((2 * E, DEN_W), jnp.float32),
        mesh=mesh,
        compiler_params=pltpu.CompilerParams(needs_layout_passes=False, use_tc_tiling_on_sc=False),
        scratch_types=[
            pltpu.VMEM((N, 2 * hsc), jnp.float32),
            pltpu.VMEM((CE, DEN_W), jnp.float32),
            pltpu.VMEM((CE,), jnp.int32),
            pltpu.VMEM((CE,), jnp.int32),
        ],
    )
    return f(src, dst, s2_flat)


def _edge_agg_call(hsc, colw, src, dst, hw_flat, ex_flat):
    mesh = plsc.VectorSubcoreMesh(core_axis_name="c", subcore_axis_name="s")
    NCB = BT // C
    NB = EPT // BT

    def body(src_hbm, dst_hbm, hw_hbm, ex_hbm, acc_hbm, den_hbm,
             acc_sh, den_sh, rows0, rows1, exb0, exb1, srcb, dstb,
             dstc0, dstc1, semg0, semg1, seme0, seme1, sems0, sems1):
        c = lax.axis_index("c")
        t = lax.axis_index("s")
        cN = c * N
        base = t * WB_ROWS
        iota16 = jnp.arange(16, dtype=jnp.int32)
        zero16 = jnp.zeros((16,), jnp.float32)
        rows = (rows0, rows1)
        exb = (exb0, exb1)
        dstc = (dstc0, dstc1)
        semg = (semg0, semg1)
        seme = (seme0, seme1)
        sems = (sems0, sems1)
        ebase = t * EPT
        exbase = c * E + ebase

        def zrow(r, _):
            for v in range(128 // 16):
                rows0[r, pl.ds(v * 16, 16)] = zero16
            exb0[r, pl.ds(0, 16)] = zero16
            return 0
        lax.fori_loop(0, C, zrow, 0)

        @pl.when(t < WB_TILES)
        def _zero():
            for j in range(WB_ROWS // WB_CH):
                pltpu.sync_copy(rows0.at[pl.ds(0, WB_CH), :],
                                acc_sh.at[pl.ds(base + j * WB_CH, WB_CH), :])
                pltpu.sync_copy(exb0.at[pl.ds(0, WB_CH), :],
                                den_sh.at[pl.ds(base + j * WB_CH, WB_CH), :])
        plsc.subcore_barrier()

        def start_chunk(b, j, s):
            off = b * BT + j * C
            pltpu.async_copy(ex_hbm.at[pl.ds(exbase + off, C), :],
                             exb[s], seme[s])
            pltpu.async_copy(hw_hbm.at[srcb.at[pl.ds(j * C, C)]],
                             rows[s], semg[s])

        def do_chunk(b, j, s):
            o = 1 - s
            pltpu.make_async_copy(hw_hbm.at[srcb.at[pl.ds(0, C)]],
                                  rows[s], semg[s]).wait()
            pltpu.make_async_copy(ex_hbm.at[pl.ds(exbase, C), :],
                                  exb[s], seme[s]).wait()

            @pl.when(j >= 1)
            def _drain_other():
                pltpu.make_async_copy(rows[o], acc_sh.at[dstc[o]],
                                      sems[o]).wait()
                pltpu.make_async_copy(exb[o], den_sh.at[dstc[o]],
                                      sems[o]).wait()

            @pl.when(j < NCB - 1)
            def _prefetch():
                start_chunk(b, j + 1, o)

            for k in range(C // 16):
                dstc[s][pl.ds(k * 16, 16)] = dstb[pl.ds(j * C + k * 16, 16)]

            def group(g, _):
                ev = g * 16 + iota16
                for h in range(hsc):
                    ex = plsc.load_gather(
                        exb[s], [ev, jnp.full((16,), h, jnp.int32)])
                    for col in range(h * colw, (h + 1) * colw):
                        cv = jnp.full((16,), col, jnp.int32)
                        r = plsc.load_gather(rows[s], [ev, cv])
                        plsc.store_scatter(rows[s], [ev, cv], r * ex)
                return 0
            lax.fori_loop(0, C // 16, group, 0)

            pltpu.async_copy(rows[s], acc_sh.at[dstc[s]], sems[s], add=True)
            pltpu.async_copy(exb[s], den_sh.at[dstc[s]], sems[s], add=True)

        for b in range(NB):
            pltpu.sync_copy(src_hbm.at[pl.ds(ebase + b * BT, BT)], srcb)
            pltpu.sync_copy(dst_hbm.at[pl.ds(ebase + b * BT, BT)], dstb)

            def adj(k, _):
                srcb[pl.ds(k * 16, 16)] = srcb[pl.ds(k * 16, 16)] + cN
                return 0
            lax.fori_loop(0, BT // 16, adj, 0)

            start_chunk(b, 0, 0)

            def inner(j, _):
                @pl.when(j % 2 == 0)
                def _even():
                    do_chunk(b, j, 0)

                @pl.when(j % 2 == 1)
                def _odd():
                    do_chunk(b, j, 1)
                return 0
            lax.fori_loop(0, NCB, inner, 0)

            s_last = (NCB - 1) % 2
            pltpu.make_async_copy(rows[s_last], acc_sh.at[dstc[s_last]],
                                  sems[s_last]).wait()
            pltpu.make_async_copy(exb[s_last], den_sh.at[dstc[s_last]],
                                  sems[s_last]).wait()

        plsc.subcore_barrier()

        @pl.when(t < WB_TILES)
        def _writeback():
            for j in range(WB_ROWS // WB_CH):
                o = base + j * WB_CH
                pltpu.sync_copy(acc_sh.at[pl.ds(o, WB_CH), :],
                                acc_hbm.at[pl.ds(cN + o, WB_CH), :])
                pltpu.sync_copy(den_sh.at[pl.ds(o, WB_CH), :],
                                den_hbm.at[pl.ds(cN + o, WB_CH), :])

    f = pl.kernel(
        body,
        out_type=(jax.ShapeDtypeStruct((2 * N, 128), jnp.float32),
                  jax.ShapeDtypeStruct((2 * N, DEN_W), jnp.float32)),
        mesh=mesh,
        compiler_params=pltpu.CompilerParams(needs_layout_passes=False, use_tc_tiling_on_sc=False),
        scratch_types=[
            pltpu.VMEM_SHARED((N, 128), jnp.float32),
            pltpu.VMEM_SHARED((N, DEN_W), jnp.float32),
            pltpu.VMEM((C, 128), jnp.float32),
            pltpu.VMEM((C, 128), jnp.float32),
            pltpu.VMEM((C, DEN_W), jnp.float32),
            pltpu.VMEM((C, DEN_W), jnp.float32),
            pltpu.VMEM((BT,), jnp.int32),
            pltpu.VMEM((BT,), jnp.int32),
            pltpu.VMEM((C,), jnp.int32),
            pltpu.VMEM((C,), jnp.int32),
            pltpu.SemaphoreType.DMA,
            pltpu.SemaphoreType.DMA,
            pltpu.SemaphoreType.DMA,
            pltpu.SemaphoreType.DMA,
            pltpu.SemaphoreType.DMA,
            pltpu.SemaphoreType.DMA,
        ],
    )
    return f(src, dst, hw_flat, ex_flat)


def _edge_sc_call(hsc, colw, src, dst, hw_flat, s2_flat):
    ex_flat = _edge_ex_call(hsc, src, dst, s2_flat)
    return _edge_agg_call(hsc, colw, src, dst, hw_flat, ex_flat)


def _ln(x, g, b, eps=1e-5):
    m = jnp.mean(x, axis=-1, keepdims=True)
    v = jnp.mean((x - m) ** 2, axis=-1, keepdims=True)
    return (x - m) / jnp.sqrt(v + eps) * g + b


def _front_body(vis, ohc, emb, w1, b1, w2, b2, g, b, out):
    h1 = jnp.maximum(jnp.dot(vis[...], w1[...],
                             preferred_element_type=jnp.float32) + b1[...], 0.0)
    v = jnp.dot(h1, w2[...], preferred_element_type=jnp.float32) + b2[...]
    out[:, 0:128] = jnp.dot(ohc[...], emb[...],
                            preferred_element_type=jnp.float32)
    out[:, 128:256] = _ln(v, g[...], b[...])


def _front(vis, ohc, emb, w1, b1, w2, b2, g, b):
    bn = 1000
    grid = (N // bn,)
    return pl.pallas_call(
        _front_body,
        grid=grid,
        in_specs=[
            pl.BlockSpec((bn, 2048), lambda i: (i, 0)),
            pl.BlockSpec((bn, 128), lambda i: (i, 0)),
            pl.BlockSpec((128, 128), lambda i: (0, 0)),
            pl.BlockSpec((2048, 512), lambda i: (0, 0)),
            pl.BlockSpec((1, 512), lambda i: (0, 0)),
            pl.BlockSpec((512, 128), lambda i: (0, 0)),
            pl.BlockSpec((1, 128), lambda i: (0, 0)),
            pl.BlockSpec((1, 128), lambda i: (0, 0)),
            pl.BlockSpec((1, 128), lambda i: (0, 0)),
        ],
        out_specs=pl.BlockSpec((bn, 256), lambda i: (i, 0)),
        out_shape=jax.ShapeDtypeStruct((N, 256), jnp.float32),
    )(vis, ohc, emb, w1, b1, w2, b2, g, b)


def _pre_body(hsc, h, w, asrc, adst, hw2, s2, exs):
    hw = jnp.dot(h[...], w[...], preferred_element_type=jnp.float32)
    ss = jnp.dot(hw, asrc[...], preferred_element_type=jnp.float32)
    sd = jnp.dot(hw, adst[...], preferred_element_type=jnp.float32)
    hw2[0] = hw[:, 0:128]
    hw2[1] = hw[:, 128:256]
    if hsc * 2 == ss.shape[1]:
        s2[0, :, 0:hsc] = ss[:, 0:hsc]
        s2[0, :, hsc:2 * hsc] = sd[:, 0:hsc]
        s2[1, :, 0:hsc] = ss[:, hsc:2 * hsc]
        s2[1, :, hsc:2 * hsc] = sd[:, hsc:2 * hsc]
    else:
        s2[0, :, 0:1] = ss
        s2[0, :, 1:2] = sd
        s2[1, :, 0:1] = ss
        s2[1, :, 1:2] = sd
    a = ss + sd
    a = jnp.where(a > 0, a, 0.2 * a)
    exs[...] = jnp.exp(a)


def _pre(h, w, asrc, adst, heads, hsc):
    bn = 1000
    grid = (N // bn,)
    return pl.pallas_call(
        functools.partial(_pre_body, hsc),
        grid=grid,
        in_specs=[
            pl.BlockSpec((bn, 256), lambda i: (i, 0)),
            pl.BlockSpec((256, 256), lambda i: (0, 0)),
            pl.BlockSpec((256, heads), lambda i: (0, 0)),
            pl.BlockSpec((256, heads), lambda i: (0, 0)),
        ],
        out_specs=[
            pl.BlockSpec((2, bn, 128), lambda i: (0, i, 0)),
            pl.BlockSpec((2, bn, 2 * hsc), lambda i: (0, i, 0)),
            pl.BlockSpec((bn, heads), lambda i: (i, 0)),
        ],
        out_shape=[
            jax.ShapeDtypeStruct((2, N, 128), jnp.float32),
            jax.ShapeDtypeStruct((2, N, 2 * hsc), jnp.float32),
            jax.ShapeDtypeStruct((N, heads), jnp.float32),
        ],
    )(h, w, asrc, adst)


def _post_body(hsc, acc, den, exs, hw2, h, rep, bias, g, b, out):
    num = jnp.concatenate([acc[0], acc[1]], axis=1)
    hwc = jnp.concatenate([hw2[0], hw2[1]], axis=1)
    e = exs[...]
    num = num + jnp.dot(e, rep[...],
                        preferred_element_type=jnp.float32) * hwc
    if hsc * 2 == e.shape[1]:
        denh = jnp.concatenate([den[0][:, 0:hsc], den[1][:, 0:hsc]], axis=1)
    else:
        denh = den[0][:, 0:1]
    d = jnp.dot(denh + e, rep[...], preferred_element_type=jnp.float32)
    xn = num / (d + 1e-16) + bias[...]
    xn = jnp.where(xn > 0, xn, jnp.exp(xn) - 1.0)
    out[...] = _ln(xn + h[...], g[...], b[...])


def _post(acc, den, exs, hw2, h, rep, bias, g, b, heads, hsc):
    bn = 1000
    grid = (N // bn,)
    return pl.pallas_call(
        functools.partial(_post_body, hsc),
        grid=grid,
        in_specs=[
            pl.BlockSpec((2, bn, 128), lambda i: (0, i, 0)),
            pl.BlockSpec((2, bn, DEN_W), lambda i: (0, i, 0)),
            pl.BlockSpec((bn, heads), lambda i: (i, 0)),
            pl.BlockSpec((2, bn, 128), lambda i: (0, i, 0)),
            pl.BlockSpec((bn, 256), lambda i: (i, 0)),
            pl.BlockSpec((heads, 256), lambda i: (0, 0)),
            pl.BlockSpec((1, 256), lambda i: (0, 0)),
            pl.BlockSpec((1, 256), lambda i: (0, 0)),
            pl.BlockSpec((1, 256), lambda i: (0, 0)),
        ],
        out_specs=pl.BlockSpec((bn, 256), lambda i: (i, 0)),
        out_shape=jax.ShapeDtypeStruct((N, 256), jnp.float32),
    )(acc, den, exs, hw2, h, rep, bias, g, b)


def _pool_body(nsteps, h2, oh, row, rob, out, psum, cnt):
    i = pl.program_id(0)

    @pl.when(i == 0)
    def _init():
        psum[...] = jnp.zeros_like(psum)
        cnt[...] = jnp.zeros_like(cnt)

    ohb = oh[...]
    psum[...] += lax.dot_general(ohb, h2[...], (((0,), (0,)), ((), ())),
                                 preferred_element_type=jnp.float32)
    cnt[...] += jnp.sum(ohb, axis=0, keepdims=True)

    @pl.when(i == nsteps - 1)
    def _fin():
        pooled = psum[...] / jnp.maximum(cnt[...], 1.0).reshape(G, 1)
        logit = jnp.dot(pooled, row[...],
                        preferred_element_type=jnp.float32) + rob[...]
        out[...] = 1.0 / (1.0 + jnp.exp(-logit))


def _pool(h2, oh, row, rob):
    bn = 1000
    nsteps = N // bn
    return pl.pallas_call(
        functools.partial(_pool_body, nsteps),
        grid=(nsteps,),
        in_specs=[
            pl.BlockSpec((bn, 256), lambda i: (i, 0)),
            pl.BlockSpec((bn, G), lambda i: (i, 0)),
            pl.BlockSpec((256, 1), lambda i: (0, 0)),
            pl.BlockSpec((1, 1), lambda i: (0, 0)),
        ],
        out_specs=pl.BlockSpec((G, 1), lambda i: (0, 0)),
        out_shape=jax.ShapeDtypeStruct((G, 1), jnp.float32),
        scratch_shapes=[
            pltpu.VMEM((G, 256), jnp.float32),
            pltpu.VMEM((1, G), jnp.float32),
        ],
    )(h2, oh, row, rob)


def _expander(a, heads, oc):
    rows = jnp.repeat(jnp.arange(heads), oc)
    mask = (rows[:, None] == jnp.arange(heads)[None, :]).astype(jnp.float32)
    return a.reshape(heads * oc, 1) * mask


def _rep(heads, colw):
    cols = jnp.arange(256) // colw
    return (jnp.arange(heads)[:, None] == cols[None, :]).astype(jnp.float32)


def _gat_layer(h, w, a_src, a_dst, bias, g, b, src, dst, heads):
    oc = HID // heads
    hsc = max(heads // 2, 1)
    colw = 128 // hsc
    asrc = _expander(a_src, heads, oc)
    adst = _expander(a_dst, heads, oc)
    hw2, s2, exs = _pre(h, w, asrc, adst, heads, hsc)
    acc, den = _edge_sc_call(hsc, colw, src, dst,
                             hw2.reshape(2 * N, 128),
                             s2.reshape(2 * N, 2 * hsc))
    rep = _rep(heads, HID // heads)
    return _post(acc.reshape(2, N, 128), den.reshape(2, N, DEN_W), exs, hw2,
                 h, rep, bias.reshape(1, 256), g.reshape(1, 256),
                 b.reshape(1, 256), heads, hsc)


def kernel(x, edge_index, batch, embed, vp_w1, vp_b1, vp_w2, vp_b2, vp_ln_g,
           vp_ln_b, w0, a_src0, a_dst0, bias0, n0_g, n0_b, w1, a_src1,
           a_dst1, bias1, n1_g, n1_b, ro_w, ro_b):
    vis = x[:, 1:]
    cat = x[:, 0:1].astype(jnp.int32)
    ohc = (cat == jnp.arange(128, dtype=jnp.int32)[None, :]).astype(
        jnp.float32)
    emb = jnp.pad(embed, ((0, 128 - embed.shape[0]), (0, 0)))
    h = _front(vis, ohc, emb, vp_w1, vp_b1.reshape(1, 512), vp_w2,
               vp_b2.reshape(1, 128), vp_ln_g.reshape(1, 128),
               vp_ln_b.reshape(1, 128))

    src = edge_index[0]
    dst = edge_index[1]
    h1 = _gat_layer(h, w0, a_src0, a_dst0, bias0, n0_g, n0_b, src, dst, 4)
    h2 = _gat_layer(h1, w1, a_src1, a_dst1, bias1, n1_g, n1_b, src, dst, 1)

    oh = (batch[:, None] == jnp.arange(G, dtype=batch.dtype)[None, :]).astype(
        jnp.float32)
    score = _pool(h2, oh, ro_w, ro_b.reshape(1, 1))
    return score.reshape(G)


# speedup vs baseline: 30.5756x; 4.2246x over previous
import functools

import jax
import jax.numpy as jnp
from jax import lax
from jax.experimental import pallas as pl
from jax.experimental.pallas import tpu as pltpu
from jax.experimental.pallas import tpu_sc as plsc

N = 10000
E = 160000
G = 64
HID = 256

NTILE = 16
EPT = E // NTILE
C = 80
BT = 2000
NCHUNK = EPT // C
WB_TILES = 10
WB_ROWS = N // WB_TILES
WB_CH = 40
DEN_W = 16


CE = 2000


def _edge_ex_call(hsc, src, dst, s2_flat):
    mesh = plsc.VectorSubcoreMesh(core_axis_name="c", subcore_axis_name="s")

    def body(src_hbm, dst_hbm, s2_hbm, ex_hbm, s_tab, exb, src_c, dst_c):
        c = lax.axis_index("c")
        t = lax.axis_index("s")
        cN = c * N
        iota16 = jnp.arange(16, dtype=jnp.int32)
        zero16 = jnp.zeros((16,), jnp.float32)

        def zrow(r, _):
            exb[r, pl.ds(0, 16)] = zero16
            return 0
        lax.fori_loop(0, CE, zrow, 0)
        pltpu.sync_copy(s2_hbm.at[pl.ds(cN, N), :], s_tab)

        def chunk(j, _):
            eb = t * EPT + j * CE
            pltpu.sync_copy(src_hbm.at[pl.ds(eb, CE)], src_c)
            pltpu.sync_copy(dst_hbm.at[pl.ds(eb, CE)], dst_c)

            def group(g, _):
                ev = g * 16 + iota16
                sv = src_c[pl.ds(g * 16, 16)]
                dv = dst_c[pl.ds(g * 16, 16)]
                for h in range(hsc):
                    hcol = jnp.full((16,), h, jnp.int32)
                    a = (plsc.load_gather(s_tab, [sv, hcol])
                         + plsc.load_gather(s_tab, [dv, hcol + hsc]))
                    a = jnp.where(a > 0, a, 0.2 * a)
                    plsc.store_scatter(exb, [ev, hcol], jnp.exp(a))
                return 0
            lax.fori_loop(0, CE // 16, group, 0)
            pltpu.sync_copy(exb, ex_hbm.at[pl.ds(c * E + eb, CE), :])
            return 0
        lax.fori_loop(0, EPT // CE, chunk, 0)

    f = pl.kernel(
        body,
        out_type=jax.ShapeDtypeStruct((2 * E, DEN_W), jnp.float32),
        mesh=mesh,
        compiler_params=pltpu.CompilerParams(needs_layout_passes=False, use_tc_tiling_on_sc=False),
        scratch_types=[
            pltpu.VMEM((N, 2 * hsc), jnp.float32),
            pltpu.VMEM((CE, DEN_W), jnp.float32),
            pltpu.VMEM((CE,), jnp.int32),
            pltpu.VMEM((CE,), jnp.int32),
        ],
    )
    return f(src, dst, s2_flat)


def _edge_agg_call(hsc, colw, src, dst, hw_flat, ex_flat):
    mesh = plsc.VectorSubcoreMesh(core_axis_name="c", subcore_axis_name="s")
    NCB = BT // C
    NB = EPT // BT

    def body(src_hbm, dst_hbm, hw_hbm, ex_hbm, acc_hbm, den_hbm,
             acc_sh, den_sh, rows0, rows1, exb0, exb1, srcb, dstb,
             dstc0, dstc1, semg0, semg1, seme0, seme1, sems0, sems1):
        c = lax.axis_index("c")
        t = lax.axis_index("s")
        cN = c * N
        base = t * WB_ROWS
        iota16 = jnp.arange(16, dtype=jnp.int32)
        zero16 = jnp.zeros((16,), jnp.float32)
        rows = (rows0, rows1)
        exb = (exb0, exb1)
        dstc = (dstc0, dstc1)
        semg = (semg0, semg1)
        seme = (seme0, seme1)
        sems = (sems0, sems1)
        ebase = t * EPT
        exbase = c * E + ebase

        def zrow(r, _):
            for v in range(128 // 16):
                rows0[r, pl.ds(v * 16, 16)] = zero16
            exb0[r, pl.ds(0, 16)] = zero16
            return 0
        lax.fori_loop(0, C, zrow, 0)

        @pl.when(t < WB_TILES)
        def _zero():
            for j in range(WB_ROWS // WB_CH):
                pltpu.sync_copy(rows0.at[pl.ds(0, WB_CH), :],
                                acc_sh.at[pl.ds(base + j * WB_CH, WB_CH), :])
                pltpu.sync_copy(exb0.at[pl.ds(0, WB_CH), :],
                                den_sh.at[pl.ds(base + j * WB_CH, WB_CH), :])
        plsc.subcore_barrier()

        def start_chunk(b, j, s):
            off = b * BT + j * C
            pltpu.async_copy(ex_hbm.at[pl.ds(exbase + off, C), :],
                             exb[s], seme[s])
            pltpu.async_copy(hw_hbm.at[srcb.at[pl.ds(j * C, C)]],
                             rows[s], semg[s])

        def do_chunk(b, j, s):
            o = 1 - s
            pltpu.make_async_copy(hw_hbm.at[srcb.at[pl.ds(0, C)]],
                                  rows[s], semg[s]).wait()
            pltpu.make_async_copy(ex_hbm.at[pl.ds(exbase, C), :],
                                  exb[s], seme[s]).wait()

            @pl.when(j >= 1)
            def _drain_other():
                pltpu.make_async_copy(rows[o], acc_sh.at[dstc[o]],
                                      sems[o]).wait()
                pltpu.make_async_copy(exb[o], den_sh.at[dstc[o]],
                                      sems[o]).wait()

            @pl.when(j < NCB - 1)
            def _prefetch():
                start_chunk(b, j + 1, o)

            for k in range(C // 16):
                dstc[s][pl.ds(k * 16, 16)] = dstb[pl.ds(j * C + k * 16, 16)]

            def edge(e, _):
                exrow = exb[s][e, pl.ds(0, 16)]
                for h in range(hsc):
                    bc = jnp.full((16,), exrow[h], jnp.float32)
                    for v in range((h * colw) // 16, ((h + 1) * colw) // 16):
                        rows[s][e, pl.ds(v * 16, 16)] = (
                            rows[s][e, pl.ds(v * 16, 16)] * bc)
                return 0
            lax.fori_loop(0, C, edge, 0)

            pltpu.async_copy(rows[s], acc_sh.at[dstc[s]], sems[s], add=True)
            pltpu.async_copy(exb[s], den_sh.at[dstc[s]], sems[s], add=True)

        for b in range(NB):
            pltpu.sync_copy(src_hbm.at[pl.ds(ebase + b * BT, BT)], srcb)
            pltpu.sync_copy(dst_hbm.at[pl.ds(ebase + b * BT, BT)], dstb)

            def adj(k, _):
                srcb[pl.ds(k * 16, 16)] = srcb[pl.ds(k * 16, 16)] + cN
                return 0
            lax.fori_loop(0, BT // 16, adj, 0)

            start_chunk(b, 0, 0)

            def inner(j, _):
                @pl.when(j % 2 == 0)
                def _even():
                    do_chunk(b, j, 0)

                @pl.when(j % 2 == 1)
                def _odd():
                    do_chunk(b, j, 1)
                return 0
            lax.fori_loop(0, NCB, inner, 0)

            s_last = (NCB - 1) % 2
            pltpu.make_async_copy(rows[s_last], acc_sh.at[dstc[s_last]],
                                  sems[s_last]).wait()
            pltpu.make_async_copy(exb[s_last], den_sh.at[dstc[s_last]],
                                  sems[s_last]).wait()

        plsc.subcore_barrier()

        @pl.when(t < WB_TILES)
        def _writeback():
            for j in range(WB_ROWS // WB_CH):
                o = base + j * WB_CH
                pltpu.sync_copy(acc_sh.at[pl.ds(o, WB_CH), :],
                                acc_hbm.at[pl.ds(cN + o, WB_CH), :])
                pltpu.sync_copy(den_sh.at[pl.ds(o, WB_CH), :],
                                den_hbm.at[pl.ds(cN + o, WB_CH), :])

    f = pl.kernel(
        body,
        out_type=(jax.ShapeDtypeStruct((2 * N, 128), jnp.float32),
                  jax.ShapeDtypeStruct((2 * N, DEN_W), jnp.float32)),
        mesh=mesh,
        compiler_params=pltpu.CompilerParams(needs_layout_passes=False, use_tc_tiling_on_sc=False),
        scratch_types=[
            pltpu.VMEM_SHARED((N, 128), jnp.float32),
            pltpu.VMEM_SHARED((N, DEN_W), jnp.float32),
            pltpu.VMEM((C, 128), jnp.float32),
            pltpu.VMEM((C, 128), jnp.float32),
            pltpu.VMEM((C, DEN_W), jnp.float32),
            pltpu.VMEM((C, DEN_W), jnp.float32),
            pltpu.VMEM((BT,), jnp.int32),
            pltpu.VMEM((BT,), jnp.int32),
            pltpu.VMEM((C,), jnp.int32),
            pltpu.VMEM((C,), jnp.int32),
            pltpu.SemaphoreType.DMA,
            pltpu.SemaphoreType.DMA,
            pltpu.SemaphoreType.DMA,
            pltpu.SemaphoreType.DMA,
            pltpu.SemaphoreType.DMA,
            pltpu.SemaphoreType.DMA,
        ],
    )
    return f(src, dst, hw_flat, ex_flat)


def _edge_sc_call(hsc, colw, src, dst, hw_flat, s2_flat):
    ex_flat = _edge_ex_call(hsc, src, dst, s2_flat)
    return _edge_agg_call(hsc, colw, src, dst, hw_flat, ex_flat)


def _ln(x, g, b, eps=1e-5):
    m = jnp.mean(x, axis=-1, keepdims=True)
    v = jnp.mean((x - m) ** 2, axis=-1, keepdims=True)
    return (x - m) / jnp.sqrt(v + eps) * g + b


def _front_body(vis, ohc, emb, w1, b1, w2, b2, g, b, out):
    h1 = jnp.maximum(jnp.dot(vis[...], w1[...],
                             preferred_element_type=jnp.float32) + b1[...], 0.0)
    v = jnp.dot(h1, w2[...], preferred_element_type=jnp.float32) + b2[...]
    out[:, 0:128] = jnp.dot(ohc[...], emb[...],
                            preferred_element_type=jnp.float32)
    out[:, 128:256] = _ln(v, g[...], b[...])


def _front(vis, ohc, emb, w1, b1, w2, b2, g, b):
    bn = 1000
    grid = (N // bn,)
    return pl.pallas_call(
        _front_body,
        grid=grid,
        in_specs=[
            pl.BlockSpec((bn, 2048), lambda i: (i, 0)),
            pl.BlockSpec((bn, 128), lambda i: (i, 0)),
            pl.BlockSpec((128, 128), lambda i: (0, 0)),
            pl.BlockSpec((2048, 512), lambda i: (0, 0)),
            pl.BlockSpec((1, 512), lambda i: (0, 0)),
            pl.BlockSpec((512, 128), lambda i: (0, 0)),
            pl.BlockSpec((1, 128), lambda i: (0, 0)),
            pl.BlockSpec((1, 128), lambda i: (0, 0)),
            pl.BlockSpec((1, 128), lambda i: (0, 0)),
        ],
        out_specs=pl.BlockSpec((bn, 256), lambda i: (i, 0)),
        out_shape=jax.ShapeDtypeStruct((N, 256), jnp.float32),
    )(vis, ohc, emb, w1, b1, w2, b2, g, b)


def _pre_body(hsc, h, w, asrc, adst, hw2, s2, exs):
    hw = jnp.dot(h[...], w[...], preferred_element_type=jnp.float32)
    ss = jnp.dot(hw, asrc[...], preferred_element_type=jnp.float32)
    sd = jnp.dot(hw, adst[...], preferred_element_type=jnp.float32)
    hw2[0] = hw[:, 0:128]
    hw2[1] = hw[:, 128:256]
    if hsc * 2 == ss.shape[1]:
        s2[0, :, 0:hsc] = ss[:, 0:hsc]
        s2[0, :, hsc:2 * hsc] = sd[:, 0:hsc]
        s2[1, :, 0:hsc] = ss[:, hsc:2 * hsc]
        s2[1, :, hsc:2 * hsc] = sd[:, hsc:2 * hsc]
    else:
        s2[0, :, 0:1] = ss
        s2[0, :, 1:2] = sd
        s2[1, :, 0:1] = ss
        s2[1, :, 1:2] = sd
    a = ss + sd
    a = jnp.where(a > 0, a, 0.2 * a)
    exs[...] = jnp.exp(a)


def _pre(h, w, asrc, adst, heads, hsc):
    bn = 1000
    grid = (N // bn,)
    return pl.pallas_call(
        functools.partial(_pre_body, hsc),
        grid=grid,
        in_specs=[
            pl.BlockSpec((bn, 256), lambda i: (i, 0)),
            pl.BlockSpec((256, 256), lambda i: (0, 0)),
            pl.BlockSpec((256, heads), lambda i: (0, 0)),
            pl.BlockSpec((256, heads), lambda i: (0, 0)),
        ],
        out_specs=[
            pl.BlockSpec((2, bn, 128), lambda i: (0, i, 0)),
            pl.BlockSpec((2, bn, 2 * hsc), lambda i: (0, i, 0)),
            pl.BlockSpec((bn, heads), lambda i: (i, 0)),
        ],
        out_shape=[
            jax.ShapeDtypeStruct((2, N, 128), jnp.float32),
            jax.ShapeDtypeStruct((2, N, 2 * hsc), jnp.float32),
            jax.ShapeDtypeStruct((N, heads), jnp.float32),
        ],
    )(h, w, asrc, adst)


def _post_body(hsc, acc, den, exs, hw2, h, rep, bias, g, b, out):
    num = jnp.concatenate([acc[0], acc[1]], axis=1)
    hwc = jnp.concatenate([hw2[0], hw2[1]], axis=1)
    e = exs[...]
    num = num + jnp.dot(e, rep[...],
                        preferred_element_type=jnp.float32) * hwc
    if hsc * 2 == e.shape[1]:
        denh = jnp.concatenate([den[0][:, 0:hsc], den[1][:, 0:hsc]], axis=1)
    else:
        denh = den[0][:, 0:1]
    d = jnp.dot(denh + e, rep[...], preferred_element_type=jnp.float32)
    xn = num / (d + 1e-16) + bias[...]
    xn = jnp.where(xn > 0, xn, jnp.exp(xn) - 1.0)
    out[...] = _ln(xn + h[...], g[...], b[...])


def _post(acc, den, exs, hw2, h, rep, bias, g, b, heads, hsc):
    bn = 1000
    grid = (N // bn,)
    return pl.pallas_call(
        functools.partial(_post_body, hsc),
        grid=grid,
        in_specs=[
            pl.BlockSpec((2, bn, 128), lambda i: (0, i, 0)),
            pl.BlockSpec((2, bn, DEN_W), lambda i: (0, i, 0)),
            pl.BlockSpec((bn, heads), lambda i: (i, 0)),
            pl.BlockSpec((2, bn, 128), lambda i: (0, i, 0)),
            pl.BlockSpec((bn, 256), lambda i: (i, 0)),
            pl.BlockSpec((heads, 256), lambda i: (0, 0)),
            pl.BlockSpec((1, 256), lambda i: (0, 0)),
            pl.BlockSpec((1, 256), lambda i: (0, 0)),
            pl.BlockSpec((1, 256), lambda i: (0, 0)),
        ],
        out_specs=pl.BlockSpec((bn, 256), lambda i: (i, 0)),
        out_shape=jax.ShapeDtypeStruct((N, 256), jnp.float32),
    )(acc, den, exs, hw2, h, rep, bias, g, b)


def _pool_body(nsteps, h2, oh, row, rob, out, psum, cnt):
    i = pl.program_id(0)

    @pl.when(i == 0)
    def _init():
        psum[...] = jnp.zeros_like(psum)
        cnt[...] = jnp.zeros_like(cnt)

    ohb = oh[...]
    psum[...] += lax.dot_general(ohb, h2[...], (((0,), (0,)), ((), ())),
                                 preferred_element_type=jnp.float32)
    cnt[...] += jnp.sum(ohb, axis=0, keepdims=True)

    @pl.when(i == nsteps - 1)
    def _fin():
        pooled = psum[...] / jnp.maximum(cnt[...], 1.0).reshape(G, 1)
        logit = jnp.dot(pooled, row[...],
                        preferred_element_type=jnp.float32) + rob[...]
        out[...] = 1.0 / (1.0 + jnp.exp(-logit))


def _pool(h2, oh, row, rob):
    bn = 1000
    nsteps = N // bn
    return pl.pallas_call(
        functools.partial(_pool_body, nsteps),
        grid=(nsteps,),
        in_specs=[
            pl.BlockSpec((bn, 256), lambda i: (i, 0)),
            pl.BlockSpec((bn, G), lambda i: (i, 0)),
            pl.BlockSpec((256, 1), lambda i: (0, 0)),
            pl.BlockSpec((1, 1), lambda i: (0, 0)),
        ],
        out_specs=pl.BlockSpec((G, 1), lambda i: (0, 0)),
        out_shape=jax.ShapeDtypeStruct((G, 1), jnp.float32),
        scratch_shapes=[
            pltpu.VMEM((G, 256), jnp.float32),
            pltpu.VMEM((1, G), jnp.float32),
        ],
    )(h2, oh, row, rob)


def _expander(a, heads, oc):
    rows = jnp.repeat(jnp.arange(heads), oc)
    mask = (rows[:, None] == jnp.arange(heads)[None, :]).astype(jnp.float32)
    return a.reshape(heads * oc, 1) * mask


def _rep(heads, colw):
    cols = jnp.arange(256) // colw
    return (jnp.arange(heads)[:, None] == cols[None, :]).astype(jnp.float32)


def _gat_layer(h, w, a_src, a_dst, bias, g, b, src, dst, heads):
    oc = HID // heads
    hsc = max(heads // 2, 1)
    colw = 128 // hsc
    asrc = _expander(a_src, heads, oc)
    adst = _expander(a_dst, heads, oc)
    hw2, s2, exs = _pre(h, w, asrc, adst, heads, hsc)
    acc, den = _edge_sc_call(hsc, colw, src, dst,
                             hw2.reshape(2 * N, 128),
                             s2.reshape(2 * N, 2 * hsc))
    rep = _rep(heads, HID // heads)
    return _post(acc.reshape(2, N, 128), den.reshape(2, N, DEN_W), exs, hw2,
                 h, rep, bias.reshape(1, 256), g.reshape(1, 256),
                 b.reshape(1, 256), heads, hsc)


def kernel(x, edge_index, batch, embed, vp_w1, vp_b1, vp_w2, vp_b2, vp_ln_g,
           vp_ln_b, w0, a_src0, a_dst0, bias0, n0_g, n0_b, w1, a_src1,
           a_dst1, bias1, n1_g, n1_b, ro_w, ro_b):
    vis = x[:, 1:]
    cat = x[:, 0:1].astype(jnp.int32)
    ohc = (cat == jnp.arange(128, dtype=jnp.int32)[None, :]).astype(
        jnp.float32)
    emb = jnp.pad(embed, ((0, 128 - embed.shape[0]), (0, 0)))
    h = _front(vis, ohc, emb, vp_w1, vp_b1.reshape(1, 512), vp_w2,
               vp_b2.reshape(1, 128), vp_ln_g.reshape(1, 128),
               vp_ln_b.reshape(1, 128))

    src = edge_index[0]
    dst = edge_index[1]
    h1 = _gat_layer(h, w0, a_src0, a_dst0, bias0, n0_g, n0_b, src, dst, 4)
    h2 = _gat_layer(h1, w1, a_src1, a_dst1, bias1, n1_g, n1_b, src, dst, 1)

    oh = (batch[:, None] == jnp.arange(G, dtype=batch.dtype)[None, :]).astype(
        jnp.float32)
    score = _pool(h2, oh, ro_w, ro_b.reshape(1, 1))
    return score.reshape(G)


# speedup vs baseline: 32.2422x; 1.0545x over previous
import functools

import jax
import jax.numpy as jnp
from jax import lax
from jax.experimental import pallas as pl
from jax.experimental.pallas import tpu as pltpu
from jax.experimental.pallas import tpu_sc as plsc

N = 10000
E = 160000
G = 64
HID = 256

NTILE = 16
EPT = E // NTILE
C = 80
BT = 2000
NCHUNK = EPT // C
WB_TILES = 10
WB_ROWS = N // WB_TILES
WB_CH = 40
DEN_W = 16


CE = 2000


def _edge_ex_call(hsc, src, dst, s2_flat):
    mesh = plsc.VectorSubcoreMesh(core_axis_name="c", subcore_axis_name="s")

    def body(src_hbm, dst_hbm, s2_hbm, ex_hbm, s_tab, exb, src_c, dst_c):
        c = lax.axis_index("c")
        t = lax.axis_index("s")
        cN = c * N
        iota16 = jnp.arange(16, dtype=jnp.int32)
        zero16 = jnp.zeros((16,), jnp.float32)

        def zrow(r, _):
            exb[r, pl.ds(0, 16)] = zero16
            return 0
        lax.fori_loop(0, CE, zrow, 0)
        pltpu.sync_copy(s2_hbm.at[pl.ds(cN, N), :], s_tab)

        def chunk(j, _):
            eb = t * EPT + j * CE
            pltpu.sync_copy(src_hbm.at[pl.ds(eb, CE)], src_c)
            pltpu.sync_copy(dst_hbm.at[pl.ds(eb, CE)], dst_c)

            def group(g, _):
                ev = g * 16 + iota16
                sv = src_c[pl.ds(g * 16, 16)]
                dv = dst_c[pl.ds(g * 16, 16)]
                for h in range(hsc):
                    hcol = jnp.full((16,), h, jnp.int32)
                    a = (plsc.load_gather(s_tab, [sv, hcol])
                         + plsc.load_gather(s_tab, [dv, hcol + hsc]))
                    a = jnp.where(a > 0, a, 0.2 * a)
                    plsc.store_scatter(exb, [ev, hcol], jnp.exp(a))
                return 0
            lax.fori_loop(0, CE // 16, group, 0)
            pltpu.sync_copy(exb, ex_hbm.at[pl.ds(c * E + eb, CE), :])
            return 0
        lax.fori_loop(0, EPT // CE, chunk, 0)

    f = pl.kernel(
        body,
        out_type=jax.ShapeDtypeStruct((2 * E, DEN_W), jnp.float32),
        mesh=mesh,
        compiler_params=pltpu.CompilerParams(needs_layout_passes=False, use_tc_tiling_on_sc=False),
        scratch_types=[
            pltpu.VMEM((N, 2 * hsc), jnp.float32),
            pltpu.VMEM((CE, DEN_W), jnp.float32),
            pltpu.VMEM((CE,), jnp.int32),
            pltpu.VMEM((CE,), jnp.int32),
        ],
    )
    return f(src, dst, s2_flat)


def _edge_agg_call(hsc, colw, src, dst, hw_flat, ex_flat):
    mesh = plsc.VectorSubcoreMesh(core_axis_name="c", subcore_axis_name="s")
    NCB = BT // C
    NB = EPT // BT

    def body(src_hbm, dst_hbm, hw_hbm, ex_hbm, acc_hbm, den_hbm,
             acc_sh, den_sh, rows0, rows1, exb0, exb1, srcb, dstb,
             dstc0, dstc1, semg0, semg1, seme0, seme1, sems0, sems1):
        c = lax.axis_index("c")
        t = lax.axis_index("s")
        cN = c * N
        base = t * WB_ROWS
        iota16 = jnp.arange(16, dtype=jnp.int32)
        zero16 = jnp.zeros((16,), jnp.float32)
        rows = (rows0, rows1)
        exb = (exb0, exb1)
        dstc = (dstc0, dstc1)
        semg = (semg0, semg1)
        seme = (seme0, seme1)
        sems = (sems0, sems1)
        ebase = t * EPT
        exbase = c * E + ebase

        def zrow(r, _):
            for v in range(128 // 16):
                rows0[r, pl.ds(v * 16, 16)] = zero16
            exb0[r, pl.ds(0, 16)] = zero16
            return 0
        lax.fori_loop(0, C, zrow, 0)

        @pl.when(t < WB_TILES)
        def _zero():
            for j in range(WB_ROWS // WB_CH):
                pltpu.sync_copy(rows0.at[pl.ds(0, WB_CH), :],
                                acc_sh.at[pl.ds(base + j * WB_CH, WB_CH), :])
                pltpu.sync_copy(exb0.at[pl.ds(0, WB_CH), :],
                                den_sh.at[pl.ds(base + j * WB_CH, WB_CH), :])
        plsc.subcore_barrier()

        def start_chunk(b, j, s):
            off = b * BT + j * C
            pltpu.async_copy(ex_hbm.at[pl.ds(exbase + off, C), :],
                             exb[s], seme[s])
            pltpu.async_copy(hw_hbm.at[srcb.at[pl.ds(j * C, C)]],
                             rows[s], semg[s])

        def do_chunk(b, j, s):
            o = 1 - s
            pltpu.make_async_copy(hw_hbm.at[srcb.at[pl.ds(0, C)]],
                                  rows[s], semg[s]).wait()
            pltpu.make_async_copy(ex_hbm.at[pl.ds(exbase, C), :],
                                  exb[s], seme[s]).wait()

            @pl.when(j >= 1)
            def _drain_other():
                pltpu.make_async_copy(rows[o], acc_sh.at[dstc[o]],
                                      sems[o]).wait()
                pltpu.make_async_copy(exb[o], den_sh.at[dstc[o]],
                                      sems[o]).wait()

            @pl.when(j < NCB - 1)
            def _prefetch():
                start_chunk(b, j + 1, o)

            for k in range(C // 16):
                dstc[s][pl.ds(k * 16, 16)] = dstb[pl.ds(j * C + k * 16, 16)]

            def edge(e, _):
                exrow = exb[s][e, pl.ds(0, 16)]
                for h in range(hsc):
                    bc = jnp.full((16,), exrow[h], jnp.float32)
                    for v in range((h * colw) // 16, ((h + 1) * colw) // 16):
                        rows[s][e, pl.ds(v * 16, 16)] = (
                            rows[s][e, pl.ds(v * 16, 16)] * bc)
                return 0
            lax.fori_loop(0, C, edge, 0)

            pltpu.async_copy(rows[s], acc_sh.at[dstc[s]], sems[s], add=True)
            pltpu.async_copy(exb[s], den_sh.at[dstc[s]], sems[s], add=True)

        for b in range(NB):
            pltpu.sync_copy(src_hbm.at[pl.ds(ebase + b * BT, BT)], srcb)
            pltpu.sync_copy(dst_hbm.at[pl.ds(ebase + b * BT, BT)], dstb)

            def adj(k, _):
                srcb[pl.ds(k * 16, 16)] = srcb[pl.ds(k * 16, 16)] + cN
                return 0
            lax.fori_loop(0, BT // 16, adj, 0)

            start_chunk(b, 0, 0)

            def inner(j, _):
                @pl.when(j % 2 == 0)
                def _even():
                    do_chunk(b, j, 0)

                @pl.when(j % 2 == 1)
                def _odd():
                    do_chunk(b, j, 1)
                return 0
            lax.fori_loop(0, NCB, inner, 0)

            s_last = (NCB - 1) % 2
            pltpu.make_async_copy(rows[s_last], acc_sh.at[dstc[s_last]],
                                  sems[s_last]).wait()
            pltpu.make_async_copy(exb[s_last], den_sh.at[dstc[s_last]],
                                  sems[s_last]).wait()

        plsc.subcore_barrier()

        @pl.when(t < WB_TILES)
        def _writeback():
            for j in range(WB_ROWS // WB_CH):
                o = base + j * WB_CH
                pltpu.sync_copy(acc_sh.at[pl.ds(o, WB_CH), :],
                                acc_hbm.at[pl.ds(cN + o, WB_CH), :])
                pltpu.sync_copy(den_sh.at[pl.ds(o, WB_CH), :],
                                den_hbm.at[pl.ds(cN + o, WB_CH), :])

    f = pl.kernel(
        body,
        out_type=(jax.ShapeDtypeStruct((2 * N, 128), jnp.float32),
                  jax.ShapeDtypeStruct((2 * N, DEN_W), jnp.float32)),
        mesh=mesh,
        compiler_params=pltpu.CompilerParams(needs_layout_passes=False, use_tc_tiling_on_sc=False),
        scratch_types=[
            pltpu.VMEM_SHARED((N, 128), jnp.float32),
            pltpu.VMEM_SHARED((N, DEN_W), jnp.float32),
            pltpu.VMEM((C, 128), jnp.float32),
            pltpu.VMEM((C, 128), jnp.float32),
            pltpu.VMEM((C, DEN_W), jnp.float32),
            pltpu.VMEM((C, DEN_W), jnp.float32),
            pltpu.VMEM((BT,), jnp.int32),
            pltpu.VMEM((BT,), jnp.int32),
            pltpu.VMEM((C,), jnp.int32),
            pltpu.VMEM((C,), jnp.int32),
            pltpu.SemaphoreType.DMA,
            pltpu.SemaphoreType.DMA,
            pltpu.SemaphoreType.DMA,
            pltpu.SemaphoreType.DMA,
            pltpu.SemaphoreType.DMA,
            pltpu.SemaphoreType.DMA,
        ],
    )
    return f(src, dst, hw_flat, ex_flat)


def _edge_sc_call(hsc, colw, src, dst, hw_flat, s2_flat):
    ex_flat = _edge_ex_call(hsc, src, dst, s2_flat)
    return _edge_agg_call(hsc, colw, src, dst, hw_flat, ex_flat)


def _ln(x, g, b, eps=1e-5):
    m = jnp.mean(x, axis=-1, keepdims=True)
    v = jnp.mean((x - m) ** 2, axis=-1, keepdims=True)
    return (x - m) / jnp.sqrt(v + eps) * g + b


def _front_body(x, emb, w1, b1, w2, b2, g, b, out):
    xr = x[...]
    ohc = (xr[:, 0:1].astype(jnp.int32) == lax.broadcasted_iota(
        jnp.int32, (1, 128), 1)).astype(jnp.float32)
    h1 = jnp.maximum(
        jnp.dot(xr.astype(jnp.bfloat16), w1[...].astype(jnp.bfloat16),
                preferred_element_type=jnp.float32) + b1[...], 0.0)
    v = jnp.dot(h1, w2[...], preferred_element_type=jnp.float32) + b2[...]
    out[:, 0:128] = jnp.dot(ohc, emb[...],
                            preferred_element_type=jnp.float32)
    out[:, 128:256] = _ln(v, g[...], b[...])


def _front(x, emb, w1p, b1, w2, b2, g, b):
    bn = 1000
    grid = (N // bn,)
    return pl.pallas_call(
        _front_body,
        grid=grid,
        in_specs=[
            pl.BlockSpec((bn, 2049), lambda i: (i, 0)),
            pl.BlockSpec((128, 128), lambda i: (0, 0)),
            pl.BlockSpec((2049, 512), lambda i: (0, 0)),
            pl.BlockSpec((1, 512), lambda i: (0, 0)),
            pl.BlockSpec((512, 128), lambda i: (0, 0)),
            pl.BlockSpec((1, 128), lambda i: (0, 0)),
            pl.BlockSpec((1, 128), lambda i: (0, 0)),
            pl.BlockSpec((1, 128), lambda i: (0, 0)),
        ],
        out_specs=pl.BlockSpec((bn, 256), lambda i: (i, 0)),
        out_shape=jax.ShapeDtypeStruct((N, 256), jnp.float32),
    )(x, emb, w1p, b1, w2, b2, g, b)


def _pre_body(hsc, h, w, asrc, adst, hw2, s2, exs):
    hw = jnp.dot(h[...], w[...], preferred_element_type=jnp.float32)
    ss = jnp.dot(hw, asrc[...], preferred_element_type=jnp.float32)
    sd = jnp.dot(hw, adst[...], preferred_element_type=jnp.float32)
    hw2[0] = hw[:, 0:128]
    hw2[1] = hw[:, 128:256]
    if hsc * 2 == ss.shape[1]:
        s2[0, :, 0:hsc] = ss[:, 0:hsc]
        s2[0, :, hsc:2 * hsc] = sd[:, 0:hsc]
        s2[1, :, 0:hsc] = ss[:, hsc:2 * hsc]
        s2[1, :, hsc:2 * hsc] = sd[:, hsc:2 * hsc]
    else:
        s2[0, :, 0:1] = ss
        s2[0, :, 1:2] = sd
        s2[1, :, 0:1] = ss
        s2[1, :, 1:2] = sd
    a = ss + sd
    a = jnp.where(a > 0, a, 0.2 * a)
    exs[...] = jnp.exp(a)


def _pre(h, w, asrc, adst, heads, hsc):
    bn = 1000
    grid = (N // bn,)
    return pl.pallas_call(
        functools.partial(_pre_body, hsc),
        grid=grid,
        in_specs=[
            pl.BlockSpec((bn, 256), lambda i: (i, 0)),
            pl.BlockSpec((256, 256), lambda i: (0, 0)),
            pl.BlockSpec((256, heads), lambda i: (0, 0)),
            pl.BlockSpec((256, heads), lambda i: (0, 0)),
        ],
        out_specs=[
            pl.BlockSpec((2, bn, 128), lambda i: (0, i, 0)),
            pl.BlockSpec((2, bn, 2 * hsc), lambda i: (0, i, 0)),
            pl.BlockSpec((bn, heads), lambda i: (i, 0)),
        ],
        out_shape=[
            jax.ShapeDtypeStruct((2, N, 128), jnp.float32),
            jax.ShapeDtypeStruct((2, N, 2 * hsc), jnp.float32),
            jax.ShapeDtypeStruct((N, heads), jnp.float32),
        ],
    )(h, w, asrc, adst)


def _post_body(hsc, acc, den, exs, hw2, h, rep, bias, g, b, out):
    num = jnp.concatenate([acc[0], acc[1]], axis=1)
    hwc = jnp.concatenate([hw2[0], hw2[1]], axis=1)
    e = exs[...]
    num = num + jnp.dot(e, rep[...],
                        preferred_element_type=jnp.float32) * hwc
    if hsc * 2 == e.shape[1]:
        denh = jnp.concatenate([den[0][:, 0:hsc], den[1][:, 0:hsc]], axis=1)
    else:
        denh = den[0][:, 0:1]
    d = jnp.dot(denh + e, rep[...], preferred_element_type=jnp.float32)
    xn = num / (d + 1e-16) + bias[...]
    xn = jnp.where(xn > 0, xn, jnp.exp(xn) - 1.0)
    out[...] = _ln(xn + h[...], g[...], b[...])


def _post(acc, den, exs, hw2, h, rep, bias, g, b, heads, hsc):
    bn = 1000
    grid = (N // bn,)
    return pl.pallas_call(
        functools.partial(_post_body, hsc),
        grid=grid,
        in_specs=[
            pl.BlockSpec((2, bn, 128), lambda i: (0, i, 0)),
            pl.BlockSpec((2, bn, DEN_W), lambda i: (0, i, 0)),
            pl.BlockSpec((bn, heads), lambda i: (i, 0)),
            pl.BlockSpec((2, bn, 128), lambda i: (0, i, 0)),
            pl.BlockSpec((bn, 256), lambda i: (i, 0)),
            pl.BlockSpec((heads, 256), lambda i: (0, 0)),
            pl.BlockSpec((1, 256), lambda i: (0, 0)),
            pl.BlockSpec((1, 256), lambda i: (0, 0)),
            pl.BlockSpec((1, 256), lambda i: (0, 0)),
        ],
        out_specs=pl.BlockSpec((bn, 256), lambda i: (i, 0)),
        out_shape=jax.ShapeDtypeStruct((N, 256), jnp.float32),
    )(acc, den, exs, hw2, h, rep, bias, g, b)


def _pool_body(nsteps, h2, bf, row, rob, out, psum, cnt):
    i = pl.program_id(0)

    @pl.when(i == 0)
    def _init():
        psum[...] = jnp.zeros_like(psum)
        cnt[...] = jnp.zeros_like(cnt)

    ohb = (bf[...].astype(jnp.int32) == lax.broadcasted_iota(
        jnp.int32, (1, G), 1)).astype(jnp.float32)
    psum[...] += lax.dot_general(ohb, h2[...], (((0,), (0,)), ((), ())),
                                 preferred_element_type=jnp.float32)
    cnt[...] += jnp.sum(ohb, axis=0, keepdims=True)

    @pl.when(i == nsteps - 1)
    def _fin():
        pooled = psum[...] / jnp.maximum(cnt[...], 1.0).reshape(G, 1)
        logit = jnp.dot(pooled, row[...],
                        preferred_element_type=jnp.float32) + rob[...]
        out[...] = 1.0 / (1.0 + jnp.exp(-logit))


def _pool(h2, bf, row, rob):
    bn = 1000
    nsteps = N // bn
    return pl.pallas_call(
        functools.partial(_pool_body, nsteps),
        grid=(nsteps,),
        in_specs=[
            pl.BlockSpec((bn, 256), lambda i: (i, 0)),
            pl.BlockSpec((bn, 1), lambda i: (i, 0)),
            pl.BlockSpec((256, 1), lambda i: (0, 0)),
            pl.BlockSpec((1, 1), lambda i: (0, 0)),
        ],
        out_specs=pl.BlockSpec((G, 1), lambda i: (0, 0)),
        out_shape=jax.ShapeDtypeStruct((G, 1), jnp.float32),
        scratch_shapes=[
            pltpu.VMEM((G, 256), jnp.float32),
            pltpu.VMEM((1, G), jnp.float32),
        ],
    )(h2, bf, row, rob)


def _expander(a, heads, oc):
    rows = jnp.repeat(jnp.arange(heads), oc)
    mask = (rows[:, None] == jnp.arange(heads)[None, :]).astype(jnp.float32)
    return a.reshape(heads * oc, 1) * mask


def _rep(heads, colw):
    cols = jnp.arange(256) // colw
    return (jnp.arange(heads)[:, None] == cols[None, :]).astype(jnp.float32)


def _gat_layer(h, w, a_src, a_dst, bias, g, b, src, dst, heads):
    oc = HID // heads
    hsc = max(heads // 2, 1)
    colw = 128 // hsc
    asrc = _expander(a_src, heads, oc)
    adst = _expander(a_dst, heads, oc)
    hw2, s2, exs = _pre(h, w, asrc, adst, heads, hsc)
    acc, den = _edge_sc_call(hsc, colw, src, dst,
                             hw2.reshape(2 * N, 128),
                             s2.reshape(2 * N, 2 * hsc))
    rep = _rep(heads, HID // heads)
    return _post(acc.reshape(2, N, 128), den.reshape(2, N, DEN_W), exs, hw2,
                 h, rep, bias.reshape(1, 256), g.reshape(1, 256),
                 b.reshape(1, 256), heads, hsc)


def kernel(x, edge_index, batch, embed, vp_w1, vp_b1, vp_w2, vp_b2, vp_ln_g,
           vp_ln_b, w0, a_src0, a_dst0, bias0, n0_g, n0_b, w1, a_src1,
           a_dst1, bias1, n1_g, n1_b, ro_w, ro_b):
    emb = jnp.pad(embed, ((0, 128 - embed.shape[0]), (0, 0)))
    w1p = jnp.concatenate([jnp.zeros((1, 512), jnp.float32), vp_w1], axis=0)
    h = _front(x, emb, w1p, vp_b1.reshape(1, 512), vp_w2,
               vp_b2.reshape(1, 128), vp_ln_g.reshape(1, 128),
               vp_ln_b.reshape(1, 128))

    src = edge_index[0]
    dst = edge_index[1]
    h1 = _gat_layer(h, w0, a_src0, a_dst0, bias0, n0_g, n0_b, src, dst, 4)
    h2 = _gat_layer(h1, w1, a_src1, a_dst1, bias1, n1_g, n1_b, src, dst, 1)

    bf = batch.astype(jnp.float32).reshape(N, 1)
    score = _pool(h2, bf, ro_w, ro_b.reshape(1, 1))
    return score.reshape(G)


# speedup vs baseline: 33.2951x; 1.0327x over previous
import functools

import jax
import jax.numpy as jnp
from jax import lax
from jax.experimental import pallas as pl
from jax.experimental.pallas import tpu as pltpu
from jax.experimental.pallas import tpu_sc as plsc

N = 10000
E = 160000
G = 64
HID = 256

NTILE = 16
EPT = E // NTILE
C = 80
BT = 2000
NCHUNK = EPT // C
WB_TILES = 10
WB_ROWS = N // WB_TILES
WB_CH = 40
DEN_W = 16


CE = 2000


def _edge_ex_call(hsc, src, dst, s2_flat):
    mesh = plsc.VectorSubcoreMesh(core_axis_name="c", subcore_axis_name="s")

    def body(src_hbm, dst_hbm, s2_hbm, ex_hbm, s_tab, exb, src_c, dst_c):
        c = lax.axis_index("c")
        t = lax.axis_index("s")
        cN = c * N
        iota16 = jnp.arange(16, dtype=jnp.int32)
        zero16 = jnp.zeros((16,), jnp.float32)

        def zrow(r, _):
            exb[r, pl.ds(0, 16)] = zero16
            return 0
        lax.fori_loop(0, CE, zrow, 0)
        pltpu.sync_copy(s2_hbm.at[pl.ds(cN, N), :], s_tab)

        def chunk(j, _):
            eb = t * EPT + j * CE
            pltpu.sync_copy(src_hbm.at[pl.ds(eb, CE)], src_c)
            pltpu.sync_copy(dst_hbm.at[pl.ds(eb, CE)], dst_c)

            def group(g, _):
                ev = g * 16 + iota16
                sv = src_c[pl.ds(g * 16, 16)]
                dv = dst_c[pl.ds(g * 16, 16)]
                for h in range(hsc):
                    hcol = jnp.full((16,), h, jnp.int32)
                    a = (plsc.load_gather(s_tab, [sv, hcol])
                         + plsc.load_gather(s_tab, [dv, hcol + hsc]))
                    a = jnp.where(a > 0, a, 0.2 * a)
                    plsc.store_scatter(exb, [ev, hcol], jnp.exp(a))
                return 0
            lax.fori_loop(0, CE // 16, group, 0)
            pltpu.sync_copy(exb, ex_hbm.at[pl.ds(c * E + eb, CE), :])
            return 0
        lax.fori_loop(0, EPT // CE, chunk, 0)

    f = pl.kernel(
        body,
        out_type=jax.ShapeDtypeStruct((2 * E, DEN_W), jnp.float32),
        mesh=mesh,
        compiler_params=pltpu.CompilerParams(needs_layout_passes=False, use_tc_tiling_on_sc=False),
        scratch_types=[
            pltpu.VMEM((N, 2 * hsc), jnp.float32),
            pltpu.VMEM((CE, DEN_W), jnp.float32),
            pltpu.VMEM((CE,), jnp.int32),
            pltpu.VMEM((CE,), jnp.int32),
        ],
    )
    return f(src, dst, s2_flat)


def _edge_agg_call(hsc, colw, src, dst, hw_flat, ex_flat):
    mesh = plsc.VectorSubcoreMesh(core_axis_name="c", subcore_axis_name="s")
    NCB = BT // C
    NB = EPT // BT

    def body(src_hbm, dst_hbm, hw_hbm, ex_hbm, acc_hbm, den_hbm,
             acc_sh, den_sh, rows0, rows1, exb0, exb1, srcb, dstb,
             dstc0, dstc1, semg0, semg1, seme0, seme1, sems0, sems1):
        c = lax.axis_index("c")
        t = lax.axis_index("s")
        cN = c * N
        base = t * WB_ROWS
        iota16 = jnp.arange(16, dtype=jnp.int32)
        zero16 = jnp.zeros((16,), jnp.float32)
        rows = (rows0, rows1)
        exb = (exb0, exb1)
        dstc = (dstc0, dstc1)
        semg = (semg0, semg1)
        seme = (seme0, seme1)
        sems = (sems0, sems1)
        ebase = t * EPT
        exbase = c * E + ebase

        def zrow(r, _):
            for v in range(128 // 16):
                rows0[r, pl.ds(v * 16, 16)] = zero16
            exb0[r, pl.ds(0, 16)] = zero16
            return 0
        lax.fori_loop(0, C, zrow, 0)

        @pl.when(t < WB_TILES)
        def _zero():
            for j in range(WB_ROWS // WB_CH):
                pltpu.sync_copy(rows0.at[pl.ds(0, WB_CH), :],
                                acc_sh.at[pl.ds(base + j * WB_CH, WB_CH), :])
                pltpu.sync_copy(exb0.at[pl.ds(0, WB_CH), :],
                                den_sh.at[pl.ds(base + j * WB_CH, WB_CH), :])
        plsc.subcore_barrier()

        def start_chunk(b, j, s):
            off = b * BT + j * C
            pltpu.async_copy(ex_hbm.at[pl.ds(exbase + off, C), :],
                             exb[s], seme[s])
            pltpu.async_copy(hw_hbm.at[srcb.at[pl.ds(j * C, C)]],
                             rows[s], semg[s])

        def do_chunk(b, j, s):
            o = 1 - s
            pltpu.make_async_copy(hw_hbm.at[srcb.at[pl.ds(0, C)]],
                                  rows[s], semg[s]).wait()
            pltpu.make_async_copy(ex_hbm.at[pl.ds(exbase, C), :],
                                  exb[s], seme[s]).wait()

            @pl.when(j >= 1)
            def _drain_other():
                pltpu.make_async_copy(rows[o], acc_sh.at[dstc[o]],
                                      sems[o]).wait()
                pltpu.make_async_copy(exb[o], den_sh.at[dstc[o]],
                                      sems[o]).wait()

            @pl.when(j < NCB - 1)
            def _prefetch():
                start_chunk(b, j + 1, o)

            for k in range(C // 16):
                dstc[s][pl.ds(k * 16, 16)] = dstb[pl.ds(j * C + k * 16, 16)]

            def edge(e, _):
                exrow = exb[s][e, pl.ds(0, 16)]
                for h in range(hsc):
                    bc = jnp.full((16,), exrow[h], jnp.float32)
                    for v in range((h * colw) // 16, ((h + 1) * colw) // 16):
                        rows[s][e, pl.ds(v * 16, 16)] = (
                            rows[s][e, pl.ds(v * 16, 16)] * bc)
                return 0
            lax.fori_loop(0, C, edge, 0)

            pltpu.async_copy(rows[s], acc_sh.at[dstc[s]], sems[s], add=True)
            pltpu.async_copy(exb[s], den_sh.at[dstc[s]], sems[s], add=True)

        for b in range(NB):
            pltpu.sync_copy(src_hbm.at[pl.ds(ebase + b * BT, BT)], srcb)
            pltpu.sync_copy(dst_hbm.at[pl.ds(ebase + b * BT, BT)], dstb)

            def adj(k, _):
                srcb[pl.ds(k * 16, 16)] = srcb[pl.ds(k * 16, 16)] + cN
                return 0
            lax.fori_loop(0, BT // 16, adj, 0)

            start_chunk(b, 0, 0)

            def inner(j, _):
                @pl.when(j % 2 == 0)
                def _even():
                    do_chunk(b, j, 0)

                @pl.when(j % 2 == 1)
                def _odd():
                    do_chunk(b, j, 1)
                return 0
            lax.fori_loop(0, NCB, inner, 0)

            s_last = (NCB - 1) % 2
            pltpu.make_async_copy(rows[s_last], acc_sh.at[dstc[s_last]],
                                  sems[s_last]).wait()
            pltpu.make_async_copy(exb[s_last], den_sh.at[dstc[s_last]],
                                  sems[s_last]).wait()

        plsc.subcore_barrier()

        @pl.when(t < WB_TILES)
        def _writeback():
            for j in range(WB_ROWS // WB_CH):
                o = base + j * WB_CH
                pltpu.sync_copy(acc_sh.at[pl.ds(o, WB_CH), :],
                                acc_hbm.at[pl.ds(cN + o, WB_CH), :])
                pltpu.sync_copy(den_sh.at[pl.ds(o, WB_CH), :],
                                den_hbm.at[pl.ds(cN + o, WB_CH), :])

    f = pl.kernel(
        body,
        out_type=(jax.ShapeDtypeStruct((2 * N, 128), jnp.float32),
                  jax.ShapeDtypeStruct((2 * N, DEN_W), jnp.float32)),
        mesh=mesh,
        compiler_params=pltpu.CompilerParams(needs_layout_passes=False, use_tc_tiling_on_sc=False),
        scratch_types=[
            pltpu.VMEM_SHARED((N, 128), jnp.float32),
            pltpu.VMEM_SHARED((N, DEN_W), jnp.float32),
            pltpu.VMEM((C, 128), jnp.float32),
            pltpu.VMEM((C, 128), jnp.float32),
            pltpu.VMEM((C, DEN_W), jnp.float32),
            pltpu.VMEM((C, DEN_W), jnp.float32),
            pltpu.VMEM((BT,), jnp.int32),
            pltpu.VMEM((BT,), jnp.int32),
            pltpu.VMEM((C,), jnp.int32),
            pltpu.VMEM((C,), jnp.int32),
            pltpu.SemaphoreType.DMA,
            pltpu.SemaphoreType.DMA,
            pltpu.SemaphoreType.DMA,
            pltpu.SemaphoreType.DMA,
            pltpu.SemaphoreType.DMA,
            pltpu.SemaphoreType.DMA,
        ],
    )
    return f(src, dst, hw_flat, ex_flat)


def _ln(x, g, b, eps=1e-5):
    m = jnp.mean(x, axis=-1, keepdims=True)
    v = jnp.mean((x - m) ** 2, axis=-1, keepdims=True)
    return (x - m) / jnp.sqrt(v + eps) * g + b


def _pre_part(hsc, hv, w, asrc, adst, hw2, s2, exs):
    hw = jnp.dot(hv, w[...], preferred_element_type=jnp.float32)
    ss = jnp.dot(hw, asrc[...], preferred_element_type=jnp.float32)
    sd = jnp.dot(hw, adst[...], preferred_element_type=jnp.float32)
    hw2[0] = hw[:, 0:128]
    hw2[1] = hw[:, 128:256]
    if hsc * 2 == ss.shape[1]:
        s2[0, :, 0:hsc] = ss[:, 0:hsc]
        s2[0, :, hsc:2 * hsc] = sd[:, 0:hsc]
        s2[1, :, 0:hsc] = ss[:, hsc:2 * hsc]
        s2[1, :, hsc:2 * hsc] = sd[:, hsc:2 * hsc]
    else:
        s2[0, :, 0:1] = ss
        s2[0, :, 1:2] = sd
        s2[1, :, 0:1] = ss
        s2[1, :, 1:2] = sd
    a = ss + sd
    a = jnp.where(a > 0, a, 0.2 * a)
    exs[...] = jnp.exp(a)


def _post_part(hsc, acc, den, exs, hw2, h, rep, bias, g, b):
    num = jnp.concatenate([acc[0], acc[1]], axis=1)
    hwc = jnp.concatenate([hw2[0], hw2[1]], axis=1)
    e = exs[...]
    num = num + jnp.dot(e, rep[...],
                        preferred_element_type=jnp.float32) * hwc
    if hsc * 2 == e.shape[1]:
        denh = jnp.concatenate([den[0][:, 0:hsc], den[1][:, 0:hsc]], axis=1)
    else:
        denh = den[0][:, 0:1]
    d = jnp.dot(denh + e, rep[...], preferred_element_type=jnp.float32)
    xn = num / (d + 1e-16) + bias[...]
    xn = jnp.where(xn > 0, xn, jnp.exp(xn) - 1.0)
    return _ln(xn + h[...], g[...], b[...])


BN = 1000


def _k_front_body(x, emb, w1, b1, w2, b2, g, b, w0, a0s, a0d,
                  h_out, hw2, s2, exs):
    xr = x[...]
    ohc = (xr[:, 0:1].astype(jnp.int32) == lax.broadcasted_iota(
        jnp.int32, (1, 128), 1)).astype(jnp.float32)
    h1 = jnp.maximum(
        jnp.dot(xr.astype(jnp.bfloat16), w1[...].astype(jnp.bfloat16),
                preferred_element_type=jnp.float32) + b1[...], 0.0)
    v = jnp.dot(h1, w2[...], preferred_element_type=jnp.float32) + b2[...]
    ce = jnp.dot(ohc, emb[...], preferred_element_type=jnp.float32)
    hv = jnp.concatenate([ce, _ln(v, g[...], b[...])], axis=1)
    h_out[...] = hv
    _pre_part(2, hv, w0, a0s, a0d, hw2, s2, exs)


def _k_front(x, emb, w1p, b1, w2, b2, g, b, w0, a0s, a0d):
    grid = (N // BN,)
    return pl.pallas_call(
        _k_front_body,
        grid=grid,
        in_specs=[
            pl.BlockSpec((BN, 2049), lambda i: (i, 0)),
            pl.BlockSpec((128, 128), lambda i: (0, 0)),
            pl.BlockSpec((2049, 512), lambda i: (0, 0)),
            pl.BlockSpec((1, 512), lambda i: (0, 0)),
            pl.BlockSpec((512, 128), lambda i: (0, 0)),
            pl.BlockSpec((1, 128), lambda i: (0, 0)),
            pl.BlockSpec((1, 128), lambda i: (0, 0)),
            pl.BlockSpec((1, 128), lambda i: (0, 0)),
            pl.BlockSpec((256, 256), lambda i: (0, 0)),
            pl.BlockSpec((256, 4), lambda i: (0, 0)),
            pl.BlockSpec((256, 4), lambda i: (0, 0)),
        ],
        out_specs=[
            pl.BlockSpec((BN, 256), lambda i: (i, 0)),
            pl.BlockSpec((2, BN, 128), lambda i: (0, i, 0)),
            pl.BlockSpec((2, BN, 4), lambda i: (0, i, 0)),
            pl.BlockSpec((BN, 4), lambda i: (i, 0)),
        ],
        out_shape=[
            jax.ShapeDtypeStruct((N, 256), jnp.float32),
            jax.ShapeDtypeStruct((2, N, 128), jnp.float32),
            jax.ShapeDtypeStruct((2, N, 4), jnp.float32),
            jax.ShapeDtypeStruct((N, 4), jnp.float32),
        ],
    )(x, emb, w1p, b1, w2, b2, g, b, w0, a0s, a0d)


def _k_mid_body(acc, den, exs0, hw20, h, rep, bias, g, b, w1, a1s, a1d,
                h1_out, hw2, s2, exs):
    h1v = _post_part(2, acc, den, exs0, hw20, h, rep, bias, g, b)
    h1_out[...] = h1v
    _pre_part(1, h1v, w1, a1s, a1d, hw2, s2, exs)


def _k_mid(acc, den, exs0, hw20, h, rep, bias, g, b, w1, a1s, a1d):
    grid = (N // BN,)
    return pl.pallas_call(
        _k_mid_body,
        grid=grid,
        in_specs=[
            pl.BlockSpec((2, BN, 128), lambda i: (0, i, 0)),
            pl.BlockSpec((2, BN, DEN_W), lambda i: (0, i, 0)),
            pl.BlockSpec((BN, 4), lambda i: (i, 0)),
            pl.BlockSpec((2, BN, 128), lambda i: (0, i, 0)),
            pl.BlockSpec((BN, 256), lambda i: (i, 0)),
            pl.BlockSpec((4, 256), lambda i: (0, 0)),
            pl.BlockSpec((1, 256), lambda i: (0, 0)),
            pl.BlockSpec((1, 256), lambda i: (0, 0)),
            pl.BlockSpec((1, 256), lambda i: (0, 0)),
            pl.BlockSpec((256, 256), lambda i: (0, 0)),
            pl.BlockSpec((256, 1), lambda i: (0, 0)),
            pl.BlockSpec((256, 1), lambda i: (0, 0)),
        ],
        out_specs=[
            pl.BlockSpec((BN, 256), lambda i: (i, 0)),
            pl.BlockSpec((2, BN, 128), lambda i: (0, i, 0)),
            pl.BlockSpec((2, BN, 2), lambda i: (0, i, 0)),
            pl.BlockSpec((BN, 1), lambda i: (i, 0)),
        ],
        out_shape=[
            jax.ShapeDtypeStruct((N, 256), jnp.float32),
            jax.ShapeDtypeStruct((2, N, 128), jnp.float32),
            jax.ShapeDtypeStruct((2, N, 2), jnp.float32),
            jax.ShapeDtypeStruct((N, 1), jnp.float32),
        ],
    )(acc, den, exs0, hw20, h, rep, bias, g, b, w1, a1s, a1d)


def _k_end_body(nsteps, acc, den, exs1, hw21, h1, rep, bias, g, b, bf,
                row, rob, out, psum, cnt):
    i = pl.program_id(0)

    @pl.when(i == 0)
    def _init():
        psum[...] = jnp.zeros_like(psum)
        cnt[...] = jnp.zeros_like(cnt)

    h2v = _post_part(1, acc, den, exs1, hw21, h1, rep, bias, g, b)
    ohb = (bf[...].astype(jnp.int32) == lax.broadcasted_iota(
        jnp.int32, (1, G), 1)).astype(jnp.float32)
    psum[...] += lax.dot_general(ohb, h2v, (((0,), (0,)), ((), ())),
                                 preferred_element_type=jnp.float32)
    cnt[...] += jnp.sum(ohb, axis=0, keepdims=True)

    @pl.when(i == nsteps - 1)
    def _fin():
        pooled = psum[...] / jnp.maximum(cnt[...], 1.0).reshape(G, 1)
        logit = jnp.dot(pooled, row[...],
                        preferred_element_type=jnp.float32) + rob[...]
        out[...] = 1.0 / (1.0 + jnp.exp(-logit))


def _k_end(acc, den, exs1, hw21, h1, rep, bias, g, b, bf, row, rob):
    nsteps = N // BN
    return pl.pallas_call(
        functools.partial(_k_end_body, nsteps),
        grid=(nsteps,),
        in_specs=[
            pl.BlockSpec((2, BN, 128), lambda i: (0, i, 0)),
            pl.BlockSpec((2, BN, DEN_W), lambda i: (0, i, 0)),
            pl.BlockSpec((BN, 1), lambda i: (i, 0)),
            pl.BlockSpec((2, BN, 128), lambda i: (0, i, 0)),
            pl.BlockSpec((BN, 256), lambda i: (i, 0)),
            pl.BlockSpec((1, 256), lambda i: (0, 0)),
            pl.BlockSpec((1, 256), lambda i: (0, 0)),
            pl.BlockSpec((1, 256), lambda i: (0, 0)),
            pl.BlockSpec((1, 256), lambda i: (0, 0)),
            pl.BlockSpec((BN, 1), lambda i: (i, 0)),
            pl.BlockSpec((256, 1), lambda i: (0, 0)),
            pl.BlockSpec((1, 1), lambda i: (0, 0)),
        ],
        out_specs=pl.BlockSpec((G, 1), lambda i: (0, 0)),
        out_shape=jax.ShapeDtypeStruct((G, 1), jnp.float32),
        scratch_shapes=[
            pltpu.VMEM((G, 256), jnp.float32),
            pltpu.VMEM((1, G), jnp.float32),
        ],
    )(acc, den, exs1, hw21, h1, rep, bias, g, b, bf, row, rob)


def _expander(a, heads, oc):
    rows = jnp.repeat(jnp.arange(heads), oc)
    mask = (rows[:, None] == jnp.arange(heads)[None, :]).astype(jnp.float32)
    return a.reshape(heads * oc, 1) * mask


def _rep(heads, colw):
    cols = jnp.arange(256) // colw
    return (jnp.arange(heads)[:, None] == cols[None, :]).astype(jnp.float32)


def kernel(x, edge_index, batch, embed, vp_w1, vp_b1, vp_w2, vp_b2, vp_ln_g,
           vp_ln_b, w0, a_src0, a_dst0, bias0, n0_g, n0_b, w1, a_src1,
           a_dst1, bias1, n1_g, n1_b, ro_w, ro_b):
    emb = jnp.pad(embed, ((0, 128 - embed.shape[0]), (0, 0)))
    w1p = jnp.concatenate([jnp.zeros((1, 512), jnp.float32), vp_w1], axis=0)
    src = edge_index[0]
    dst = edge_index[1]
    bf = batch.astype(jnp.float32).reshape(N, 1)

    h, hw20, s20, exs0 = _k_front(
        x, emb, w1p, vp_b1.reshape(1, 512), vp_w2, vp_b2.reshape(1, 128),
        vp_ln_g.reshape(1, 128), vp_ln_b.reshape(1, 128), w0,
        _expander(a_src0, 4, 64), _expander(a_dst0, 4, 64))
    ex0 = _edge_ex_call(2, src, dst, s20.reshape(2 * N, 4))
    acc0, den0 = _edge_agg_call(2, 64, src, dst,
                                hw20.reshape(2 * N, 128), ex0)

    h1, hw21, s21, exs1 = _k_mid(
        acc0.reshape(2, N, 128), den0.reshape(2, N, DEN_W), exs0, hw20, h,
        _rep(4, 64), bias0.reshape(1, 256), n0_g.reshape(1, 256),
        n0_b.reshape(1, 256), w1, _expander(a_src1, 1, 256),
        _expander(a_dst1, 1, 256))
    ex1 = _edge_ex_call(1, src, dst, s21.reshape(2 * N, 2))
    acc1, den1 = _edge_agg_call(1, 128, src, dst,
                                hw21.reshape(2 * N, 128), ex1)

    score = _k_end(
        acc1.reshape(2, N, 128), den1.reshape(2, N, DEN_W), exs1, hw21, h1,
        _rep(1, 256), bias1.reshape(1, 256), n1_g.reshape(1, 256),
        n1_b.reshape(1, 256), bf, ro_w, ro_b.reshape(1, 1))
    return score.reshape(G)


# speedup vs baseline: 36.9245x; 1.1090x over previous
import functools

import jax
import jax.numpy as jnp
from jax import lax
from jax.experimental import pallas as pl
from jax.experimental.pallas import tpu as pltpu
from jax.experimental.pallas import tpu_sc as plsc

N = 10000
E = 160000
G = 64
HID = 256

NTILE = 16
EPT = E // NTILE
C = 80
BT = 2000
NCHUNK = EPT // C
WB_TILES = 10
WB_ROWS = N // WB_TILES
WB_CH = 40
DEN_W = 16


CE = 2000


def _edge_agg_call(hsc, colw, src, dst, hw_flat, s2_flat):
    mesh = plsc.VectorSubcoreMesh(core_axis_name="c", subcore_axis_name="s")
    NCB = BT // C
    NB = EPT // BT

    def body(src_hbm, dst_hbm, hw_hbm, s2_hbm, acc_hbm, den_hbm,
             acc_sh, den_sh, rows0, rows1, exb0, exb1, sbs0, sbs1,
             sbd0, sbd1, srcb, dstb, dstob, dstc0, dstc1,
             semg0, semg1, sems0, sems1):
        c = lax.axis_index("c")
        t = lax.axis_index("s")
        cN = c * N
        base = t * WB_ROWS
        iota16 = jnp.arange(16, dtype=jnp.int32)
        zero16 = jnp.zeros((16,), jnp.float32)
        rows = (rows0, rows1)
        exb = (exb0, exb1)
        sbs = (sbs0, sbs1)
        sbd = (sbd0, sbd1)
        dstc = (dstc0, dstc1)
        semg = (semg0, semg1)
        sems = (sems0, sems1)
        ebase = t * EPT

        def zrow(r, _):
            for v in range(128 // 16):
                rows0[r, pl.ds(v * 16, 16)] = zero16
            exb0[r, pl.ds(0, 16)] = zero16
            exb1[r, pl.ds(0, 16)] = zero16
            return 0
        lax.fori_loop(0, C, zrow, 0)

        @pl.when(t < WB_TILES)
        def _zero():
            for j in range(WB_ROWS // WB_CH):
                pltpu.sync_copy(rows0.at[pl.ds(0, WB_CH), :],
                                acc_sh.at[pl.ds(base + j * WB_CH, WB_CH), :])
                pltpu.sync_copy(exb0.at[pl.ds(0, WB_CH), :],
                                den_sh.at[pl.ds(base + j * WB_CH, WB_CH), :])
        plsc.subcore_barrier()

        def start_chunk(j, s):
            pltpu.async_copy(hw_hbm.at[srcb.at[pl.ds(j * C, C)]],
                             rows[s], semg[s])
            pltpu.async_copy(s2_hbm.at[srcb.at[pl.ds(j * C, C)]],
                             sbs[s], semg[s])
            pltpu.async_copy(s2_hbm.at[dstob.at[pl.ds(j * C, C)]],
                             sbd[s], semg[s])

        def do_chunk(j, s):
            o = 1 - s
            pltpu.make_async_copy(hw_hbm.at[srcb.at[pl.ds(0, C)]],
                                  rows[s], semg[s]).wait()
            pltpu.make_async_copy(s2_hbm.at[srcb.at[pl.ds(0, C)]],
                                  sbs[s], semg[s]).wait()
            pltpu.make_async_copy(s2_hbm.at[dstob.at[pl.ds(0, C)]],
                                  sbd[s], semg[s]).wait()

            @pl.when(j >= 1)
            def _drain_other():
                pltpu.make_async_copy(rows[o], acc_sh.at[dstc[o]],
                                      sems[o]).wait()
                pltpu.make_async_copy(exb[o], den_sh.at[dstc[o]],
                                      sems[o]).wait()

            @pl.when(j < NCB - 1)
            def _prefetch():
                start_chunk(j + 1, o)

            for k in range(C // 16):
                dstc[s][pl.ds(k * 16, 16)] = dstb[pl.ds(j * C + k * 16, 16)]

            def group(g, _):
                ev = g * 16 + iota16
                for h in range(hsc):
                    hcol = jnp.full((16,), h, jnp.int32)
                    a = (plsc.load_gather(sbs[s], [ev, hcol])
                         + plsc.load_gather(sbd[s], [ev, hcol + hsc]))
                    a = jnp.where(a > 0, a, 0.2 * a)
                    plsc.store_scatter(exb[s], [ev, hcol], jnp.exp(a))
                return 0
            lax.fori_loop(0, C // 16, group, 0)

            def edge(e, _):
                exrow = exb[s][e, pl.ds(0, 16)]
                for h in range(hsc):
                    bc = jnp.full((16,), exrow[h], jnp.float32)
                    for v in range((h * colw) // 16, ((h + 1) * colw) // 16):
                        rows[s][e, pl.ds(v * 16, 16)] = (
                            rows[s][e, pl.ds(v * 16, 16)] * bc)
                return 0
            lax.fori_loop(0, C, edge, 0)

            pltpu.async_copy(rows[s], acc_sh.at[dstc[s]], sems[s], add=True)
            pltpu.async_copy(exb[s], den_sh.at[dstc[s]], sems[s], add=True)

        for b in range(NB):
            pltpu.sync_copy(src_hbm.at[pl.ds(ebase + b * BT, BT)], srcb)
            pltpu.sync_copy(dst_hbm.at[pl.ds(ebase + b * BT, BT)], dstb)

            def adj(k, _):
                srcb[pl.ds(k * 16, 16)] = srcb[pl.ds(k * 16, 16)] + cN
                dstob[pl.ds(k * 16, 16)] = dstb[pl.ds(k * 16, 16)] + cN
                return 0
            lax.fori_loop(0, BT // 16, adj, 0)

            start_chunk(0, 0)

            def inner(j, _):
                @pl.when(j % 2 == 0)
                def _even():
                    do_chunk(j, 0)

                @pl.when(j % 2 == 1)
                def _odd():
                    do_chunk(j, 1)
                return 0
            lax.fori_loop(0, NCB, inner, 0)

            s_last = (NCB - 1) % 2
            pltpu.make_async_copy(rows[s_last], acc_sh.at[dstc[s_last]],
                                  sems[s_last]).wait()
            pltpu.make_async_copy(exb[s_last], den_sh.at[dstc[s_last]],
                                  sems[s_last]).wait()

        plsc.subcore_barrier()

        @pl.when(t < WB_TILES)
        def _writeback():
            for j in range(WB_ROWS // WB_CH):
                o = base + j * WB_CH
                pltpu.sync_copy(acc_sh.at[pl.ds(o, WB_CH), :],
                                acc_hbm.at[pl.ds(cN + o, WB_CH), :])
                pltpu.sync_copy(den_sh.at[pl.ds(o, WB_CH), :],
                                den_hbm.at[pl.ds(cN + o, WB_CH), :])

    f = pl.kernel(
        body,
        out_type=(jax.ShapeDtypeStruct((2 * N, 128), jnp.float32),
                  jax.ShapeDtypeStruct((2 * N, DEN_W), jnp.float32)),
        mesh=mesh,
        compiler_params=pltpu.CompilerParams(needs_layout_passes=False, use_tc_tiling_on_sc=False),
        scratch_types=[
            pltpu.VMEM_SHARED((N, 128), jnp.float32),
            pltpu.VMEM_SHARED((N, DEN_W), jnp.float32),
            pltpu.VMEM((C, 128), jnp.float32),
            pltpu.VMEM((C, 128), jnp.float32),
            pltpu.VMEM((C, DEN_W), jnp.float32),
            pltpu.VMEM((C, DEN_W), jnp.float32),
            pltpu.VMEM((C, 2 * hsc), jnp.float32),
            pltpu.VMEM((C, 2 * hsc), jnp.float32),
            pltpu.VMEM((C, 2 * hsc), jnp.float32),
            pltpu.VMEM((C, 2 * hsc), jnp.float32),
            pltpu.VMEM((BT,), jnp.int32),
            pltpu.VMEM((BT,), jnp.int32),
            pltpu.VMEM((BT,), jnp.int32),
            pltpu.VMEM((C,), jnp.int32),
            pltpu.VMEM((C,), jnp.int32),
            pltpu.SemaphoreType.DMA,
            pltpu.SemaphoreType.DMA,
            pltpu.SemaphoreType.DMA,
            pltpu.SemaphoreType.DMA,
        ],
    )
    return f(src, dst, hw_flat, s2_flat)


def _ln(x, g, b, eps=1e-5):
    m = jnp.mean(x, axis=-1, keepdims=True)
    v = jnp.mean((x - m) ** 2, axis=-1, keepdims=True)
    return (x - m) / jnp.sqrt(v + eps) * g + b


def _pre_part(hsc, hv, w, asrc, adst, hw2, s2, exs):
    hw = jnp.dot(hv, w[...], preferred_element_type=jnp.float32)
    ss = jnp.dot(hw, asrc[...], preferred_element_type=jnp.float32)
    sd = jnp.dot(hw, adst[...], preferred_element_type=jnp.float32)
    hw2[0] = hw[:, 0:128]
    hw2[1] = hw[:, 128:256]
    if hsc * 2 == ss.shape[1]:
        s2[0, :, 0:hsc] = ss[:, 0:hsc]
        s2[0, :, hsc:2 * hsc] = sd[:, 0:hsc]
        s2[1, :, 0:hsc] = ss[:, hsc:2 * hsc]
        s2[1, :, hsc:2 * hsc] = sd[:, hsc:2 * hsc]
    else:
        s2[0, :, 0:1] = ss
        s2[0, :, 1:2] = sd
        s2[1, :, 0:1] = ss
        s2[1, :, 1:2] = sd
    a = ss + sd
    a = jnp.where(a > 0, a, 0.2 * a)
    exs[...] = jnp.exp(a)


def _post_part(hsc, acc, den, exs, hw2, h, rep, bias, g, b):
    num = jnp.concatenate([acc[0], acc[1]], axis=1)
    hwc = jnp.concatenate([hw2[0], hw2[1]], axis=1)
    e = exs[...]
    num = num + jnp.dot(e, rep[...],
                        preferred_element_type=jnp.float32) * hwc
    if hsc * 2 == e.shape[1]:
        denh = jnp.concatenate([den[0][:, 0:hsc], den[1][:, 0:hsc]], axis=1)
    else:
        denh = den[0][:, 0:1]
    d = jnp.dot(denh + e, rep[...], preferred_element_type=jnp.float32)
    xn = num / (d + 1e-16) + bias[...]
    xn = jnp.where(xn > 0, xn, jnp.exp(xn) - 1.0)
    return _ln(xn + h[...], g[...], b[...])


BN = 1000


def _k_front_body(x, emb, w1, b1, w2, b2, g, b, w0, a0s, a0d,
                  h_out, hw2, s2, exs):
    xr = x[...]
    ohc = (xr[:, 0:1].astype(jnp.int32) == lax.broadcasted_iota(
        jnp.int32, (1, 128), 1)).astype(jnp.float32)
    h1 = jnp.maximum(
        jnp.dot(xr.astype(jnp.bfloat16), w1[...].astype(jnp.bfloat16),
                preferred_element_type=jnp.float32) + b1[...], 0.0)
    v = jnp.dot(h1, w2[...], preferred_element_type=jnp.float32) + b2[...]
    ce = jnp.dot(ohc, emb[...], preferred_element_type=jnp.float32)
    hv = jnp.concatenate([ce, _ln(v, g[...], b[...])], axis=1)
    h_out[...] = hv
    _pre_part(2, hv, w0, a0s, a0d, hw2, s2, exs)


def _k_front(x, emb, w1p, b1, w2, b2, g, b, w0, a0s, a0d):
    grid = (N // BN,)
    return pl.pallas_call(
        _k_front_body,
        grid=grid,
        in_specs=[
            pl.BlockSpec((BN, 2049), lambda i: (i, 0)),
            pl.BlockSpec((128, 128), lambda i: (0, 0)),
            pl.BlockSpec((2049, 512), lambda i: (0, 0)),
            pl.BlockSpec((1, 512), lambda i: (0, 0)),
            pl.BlockSpec((512, 128), lambda i: (0, 0)),
            pl.BlockSpec((1, 128), lambda i: (0, 0)),
            pl.BlockSpec((1, 128), lambda i: (0, 0)),
            pl.BlockSpec((1, 128), lambda i: (0, 0)),
            pl.BlockSpec((256, 256), lambda i: (0, 0)),
            pl.BlockSpec((256, 4), lambda i: (0, 0)),
            pl.BlockSpec((256, 4), lambda i: (0, 0)),
        ],
        out_specs=[
            pl.BlockSpec((BN, 256), lambda i: (i, 0)),
            pl.BlockSpec((2, BN, 128), lambda i: (0, i, 0)),
            pl.BlockSpec((2, BN, 4), lambda i: (0, i, 0)),
            pl.BlockSpec((BN, 4), lambda i: (i, 0)),
        ],
        out_shape=[
            jax.ShapeDtypeStruct((N, 256), jnp.float32),
            jax.ShapeDtypeStruct((2, N, 128), jnp.float32),
            jax.ShapeDtypeStruct((2, N, 4), jnp.float32),
            jax.ShapeDtypeStruct((N, 4), jnp.float32),
        ],
    )(x, emb, w1p, b1, w2, b2, g, b, w0, a0s, a0d)


def _k_mid_body(acc, den, exs0, hw20, h, rep, bias, g, b, w1, a1s, a1d,
                h1_out, hw2, s2, exs):
    h1v = _post_part(2, acc, den, exs0, hw20, h, rep, bias, g, b)
    h1_out[...] = h1v
    _pre_part(1, h1v, w1, a1s, a1d, hw2, s2, exs)


def _k_mid(acc, den, exs0, hw20, h, rep, bias, g, b, w1, a1s, a1d):
    grid = (N // BN,)
    return pl.pallas_call(
        _k_mid_body,
        grid=grid,
        in_specs=[
            pl.BlockSpec((2, BN, 128), lambda i: (0, i, 0)),
            pl.BlockSpec((2, BN, DEN_W), lambda i: (0, i, 0)),
            pl.BlockSpec((BN, 4), lambda i: (i, 0)),
            pl.BlockSpec((2, BN, 128), lambda i: (0, i, 0)),
            pl.BlockSpec((BN, 256), lambda i: (i, 0)),
            pl.BlockSpec((4, 256), lambda i: (0, 0)),
            pl.BlockSpec((1, 256), lambda i: (0, 0)),
            pl.BlockSpec((1, 256), lambda i: (0, 0)),
            pl.BlockSpec((1, 256), lambda i: (0, 0)),
            pl.BlockSpec((256, 256), lambda i: (0, 0)),
            pl.BlockSpec((256, 1), lambda i: (0, 0)),
            pl.BlockSpec((256, 1), lambda i: (0, 0)),
        ],
        out_specs=[
            pl.BlockSpec((BN, 256), lambda i: (i, 0)),
            pl.BlockSpec((2, BN, 128), lambda i: (0, i, 0)),
            pl.BlockSpec((2, BN, 2), lambda i: (0, i, 0)),
            pl.BlockSpec((BN, 1), lambda i: (i, 0)),
        ],
        out_shape=[
            jax.ShapeDtypeStruct((N, 256), jnp.float32),
            jax.ShapeDtypeStruct((2, N, 128), jnp.float32),
            jax.ShapeDtypeStruct((2, N, 2), jnp.float32),
            jax.ShapeDtypeStruct((N, 1), jnp.float32),
        ],
    )(acc, den, exs0, hw20, h, rep, bias, g, b, w1, a1s, a1d)


def _k_end_body(nsteps, acc, den, exs1, hw21, h1, rep, bias, g, b, bf,
                row, rob, out, psum, cnt):
    i = pl.program_id(0)

    @pl.when(i == 0)
    def _init():
        psum[...] = jnp.zeros_like(psum)
        cnt[...] = jnp.zeros_like(cnt)

    h2v = _post_part(1, acc, den, exs1, hw21, h1, rep, bias, g, b)
    ohb = (bf[...].astype(jnp.int32) == lax.broadcasted_iota(
        jnp.int32, (1, G), 1)).astype(jnp.float32)
    psum[...] += lax.dot_general(ohb, h2v, (((0,), (0,)), ((), ())),
                                 preferred_element_type=jnp.float32)
    cnt[...] += jnp.sum(ohb, axis=0, keepdims=True)

    @pl.when(i == nsteps - 1)
    def _fin():
        pooled = psum[...] / jnp.maximum(cnt[...], 1.0).reshape(G, 1)
        logit = jnp.dot(pooled, row[...],
                        preferred_element_type=jnp.float32) + rob[...]
        out[...] = 1.0 / (1.0 + jnp.exp(-logit))


def _k_end(acc, den, exs1, hw21, h1, rep, bias, g, b, bf, row, rob):
    nsteps = N // BN
    return pl.pallas_call(
        functools.partial(_k_end_body, nsteps),
        grid=(nsteps,),
        in_specs=[
            pl.BlockSpec((2, BN, 128), lambda i: (0, i, 0)),
            pl.BlockSpec((2, BN, DEN_W), lambda i: (0, i, 0)),
            pl.BlockSpec((BN, 1), lambda i: (i, 0)),
            pl.BlockSpec((2, BN, 128), lambda i: (0, i, 0)),
            pl.BlockSpec((BN, 256), lambda i: (i, 0)),
            pl.BlockSpec((1, 256), lambda i: (0, 0)),
            pl.BlockSpec((1, 256), lambda i: (0, 0)),
            pl.BlockSpec((1, 256), lambda i: (0, 0)),
            pl.BlockSpec((1, 256), lambda i: (0, 0)),
            pl.BlockSpec((BN, 1), lambda i: (i, 0)),
            pl.BlockSpec((256, 1), lambda i: (0, 0)),
            pl.BlockSpec((1, 1), lambda i: (0, 0)),
        ],
        out_specs=pl.BlockSpec((G, 1), lambda i: (0, 0)),
        out_shape=jax.ShapeDtypeStruct((G, 1), jnp.float32),
        scratch_shapes=[
            pltpu.VMEM((G, 256), jnp.float32),
            pltpu.VMEM((1, G), jnp.float32),
        ],
    )(acc, den, exs1, hw21, h1, rep, bias, g, b, bf, row, rob)


def _expander(a, heads, oc):
    rows = jnp.repeat(jnp.arange(heads), oc)
    mask = (rows[:, None] == jnp.arange(heads)[None, :]).astype(jnp.float32)
    return a.reshape(heads * oc, 1) * mask


def _rep(heads, colw):
    cols = jnp.arange(256) // colw
    return (jnp.arange(heads)[:, None] == cols[None, :]).astype(jnp.float32)


def kernel(x, edge_index, batch, embed, vp_w1, vp_b1, vp_w2, vp_b2, vp_ln_g,
           vp_ln_b, w0, a_src0, a_dst0, bias0, n0_g, n0_b, w1, a_src1,
           a_dst1, bias1, n1_g, n1_b, ro_w, ro_b):
    emb = jnp.pad(embed, ((0, 128 - embed.shape[0]), (0, 0)))
    w1p = jnp.concatenate([jnp.zeros((1, 512), jnp.float32), vp_w1], axis=0)
    src = edge_index[0]
    dst = edge_index[1]
    bf = batch.astype(jnp.float32).reshape(N, 1)

    h, hw20, s20, exs0 = _k_front(
        x, emb, w1p, vp_b1.reshape(1, 512), vp_w2, vp_b2.reshape(1, 128),
        vp_ln_g.reshape(1, 128), vp_ln_b.reshape(1, 128), w0,
        _expander(a_src0, 4, 64), _expander(a_dst0, 4, 64))
    acc0, den0 = _edge_agg_call(2, 64, src, dst,
                                hw20.reshape(2 * N, 128),
                                s20.reshape(2 * N, 4))

    h1, hw21, s21, exs1 = _k_mid(
        acc0.reshape(2, N, 128), den0.reshape(2, N, DEN_W), exs0, hw20, h,
        _rep(4, 64), bias0.reshape(1, 256), n0_g.reshape(1, 256),
        n0_b.reshape(1, 256), w1, _expander(a_src1, 1, 256),
        _expander(a_dst1, 1, 256))
    acc1, den1 = _edge_agg_call(1, 128, src, dst,
                                hw21.reshape(2 * N, 128),
                                s21.reshape(2 * N, 2))

    score = _k_end(
        acc1.reshape(2, N, 128), den1.reshape(2, N, DEN_W), exs1, hw21, h1,
        _rep(1, 256), bias1.reshape(1, 256), n1_g.reshape(1, 256),
        n1_b.reshape(1, 256), bf, ro_w, ro_b.reshape(1, 1))
    return score.reshape(G)


# speedup vs baseline: 37.0394x; 1.0031x over previous
import functools

import jax
import jax.numpy as jnp
from jax import lax
from jax.experimental import pallas as pl
from jax.experimental.pallas import tpu as pltpu
from jax.experimental.pallas import tpu_sc as plsc

N = 10000
E = 160000
G = 64
HID = 256

NTILE = 16
EPT = E // NTILE
C = 80
BT = 2000
NCHUNK = EPT // C
WB_TILES = 10
WB_ROWS = N // WB_TILES
WB_CH = 40
DEN_W = 16


CE = 2000


def _edge_agg_call(hsc, colw, src, dst, hw_flat, s2_flat):
    mesh = plsc.VectorSubcoreMesh(core_axis_name="c", subcore_axis_name="s")
    NCB = BT // C
    NB = EPT // BT

    def body(src_hbm, dst_hbm, hw_hbm, s2_hbm, acc_hbm, den_hbm,
             acc_sh, den_sh, rows0, rows1, exb0, exb1, sbs0, sbs1,
             sbd0, sbd1, srcb, dstb, dstob, dstc0, dstc1,
             semg0, semg1, sems0, sems1):
        c = lax.axis_index("c")
        t = lax.axis_index("s")
        cN = c * N
        base = t * WB_ROWS
        iota16 = jnp.arange(16, dtype=jnp.int32)
        zero16 = jnp.zeros((16,), jnp.float32)
        rows = (rows0, rows1)
        exb = (exb0, exb1)
        sbs = (sbs0, sbs1)
        sbd = (sbd0, sbd1)
        dstc = (dstc0, dstc1)
        semg = (semg0, semg1)
        sems = (sems0, sems1)
        ebase = t * EPT

        def zrow(r, _):
            for v in range(128 // 16):
                rows0[r, pl.ds(v * 16, 16)] = zero16
            exb0[r, pl.ds(0, 16)] = zero16
            exb1[r, pl.ds(0, 16)] = zero16
            return 0
        lax.fori_loop(0, C, zrow, 0)

        @pl.when(t < WB_TILES)
        def _zero():
            for j in range(WB_ROWS // WB_CH):
                pltpu.sync_copy(rows0.at[pl.ds(0, WB_CH), :],
                                acc_sh.at[pl.ds(base + j * WB_CH, WB_CH), :])
                pltpu.sync_copy(exb0.at[pl.ds(0, WB_CH), :],
                                den_sh.at[pl.ds(base + j * WB_CH, WB_CH), :])
        plsc.subcore_barrier()

        def start_chunk(j, s):
            pltpu.async_copy(hw_hbm.at[srcb.at[pl.ds(j * C, C)]],
                             rows[s], semg[s])
            pltpu.async_copy(s2_hbm.at[srcb.at[pl.ds(j * C, C)]],
                             sbs[s], semg[s])
            pltpu.async_copy(s2_hbm.at[dstob.at[pl.ds(j * C, C)]],
                             sbd[s], semg[s])

        def do_chunk(j, s):
            o = 1 - s
            pltpu.make_async_copy(hw_hbm.at[srcb.at[pl.ds(0, C)]],
                                  rows[s], semg[s]).wait()
            pltpu.make_async_copy(s2_hbm.at[srcb.at[pl.ds(0, C)]],
                                  sbs[s], semg[s]).wait()
            pltpu.make_async_copy(s2_hbm.at[dstob.at[pl.ds(0, C)]],
                                  sbd[s], semg[s]).wait()

            @pl.when(j >= 1)
            def _drain_other():
                pltpu.make_async_copy(rows[o], acc_sh.at[dstc[o]],
                                      sems[o]).wait()
                pltpu.make_async_copy(exb[o], den_sh.at[dstc[o]],
                                      sems[o]).wait()

            @pl.when(j < NCB - 1)
            def _prefetch():
                start_chunk(j + 1, o)

            for k in range(C // 16):
                dstc[s][pl.ds(k * 16, 16)] = dstb[pl.ds(j * C + k * 16, 16)]

            def group(g, _):
                ev = g * 16 + iota16
                for h in range(hsc):
                    hcol = jnp.full((16,), h, jnp.int32)
                    a = (plsc.load_gather(sbs[s], [ev, hcol])
                         + plsc.load_gather(sbd[s], [ev, hcol + hsc]))
                    a = jnp.where(a > 0, a, 0.2 * a)
                    plsc.store_scatter(exb[s], [ev, hcol], jnp.exp(a))
                return 0
            lax.fori_loop(0, C // 16, group, 0)

            def edge(e, _):
                exrow = exb[s][e, pl.ds(0, 16)]
                for h in range(hsc):
                    bc = jnp.full((16,), exrow[h], jnp.float32)
                    for v in range((h * colw) // 16, ((h + 1) * colw) // 16):
                        rows[s][e, pl.ds(v * 16, 16)] = (
                            rows[s][e, pl.ds(v * 16, 16)] * bc)
                return 0
            lax.fori_loop(0, C, edge, 0)

            pltpu.async_copy(rows[s], acc_sh.at[dstc[s]], sems[s], add=True)
            pltpu.async_copy(exb[s], den_sh.at[dstc[s]], sems[s], add=True)

        for b in range(NB):
            pltpu.sync_copy(src_hbm.at[pl.ds(ebase + b * BT, BT)], srcb)
            pltpu.sync_copy(dst_hbm.at[pl.ds(ebase + b * BT, BT)], dstb)

            def adj(k, _):
                srcb[pl.ds(k * 16, 16)] = srcb[pl.ds(k * 16, 16)] + cN
                dstob[pl.ds(k * 16, 16)] = dstb[pl.ds(k * 16, 16)] + cN
                return 0
            lax.fori_loop(0, BT // 16, adj, 0)

            start_chunk(0, 0)

            def inner(j, _):
                @pl.when(j % 2 == 0)
                def _even():
                    do_chunk(j, 0)

                @pl.when(j % 2 == 1)
                def _odd():
                    do_chunk(j, 1)
                return 0
            lax.fori_loop(0, NCB, inner, 0)

            s_last = (NCB - 1) % 2
            pltpu.make_async_copy(rows[s_last], acc_sh.at[dstc[s_last]],
                                  sems[s_last]).wait()
            pltpu.make_async_copy(exb[s_last], den_sh.at[dstc[s_last]],
                                  sems[s_last]).wait()

        plsc.subcore_barrier()

        @pl.when(t < WB_TILES)
        def _writeback():
            for j in range(WB_ROWS // WB_CH):
                o = base + j * WB_CH
                pltpu.sync_copy(acc_sh.at[pl.ds(o, WB_CH), :],
                                acc_hbm.at[pl.ds(cN + o, WB_CH), :])
                pltpu.sync_copy(den_sh.at[pl.ds(o, WB_CH), :],
                                den_hbm.at[pl.ds(cN + o, WB_CH), :])

    f = pl.kernel(
        body,
        out_type=(jax.ShapeDtypeStruct((2 * N, 128), jnp.float32),
                  jax.ShapeDtypeStruct((2 * N, DEN_W), jnp.float32)),
        mesh=mesh,
        compiler_params=pltpu.CompilerParams(needs_layout_passes=False, use_tc_tiling_on_sc=False),
        scratch_types=[
            pltpu.VMEM_SHARED((N, 128), jnp.float32),
            pltpu.VMEM_SHARED((N, DEN_W), jnp.float32),
            pltpu.VMEM((C, 128), jnp.float32),
            pltpu.VMEM((C, 128), jnp.float32),
            pltpu.VMEM((C, DEN_W), jnp.float32),
            pltpu.VMEM((C, DEN_W), jnp.float32),
            pltpu.VMEM((C, 2 * hsc), jnp.float32),
            pltpu.VMEM((C, 2 * hsc), jnp.float32),
            pltpu.VMEM((C, 2 * hsc), jnp.float32),
            pltpu.VMEM((C, 2 * hsc), jnp.float32),
            pltpu.VMEM((BT,), jnp.int32),
            pltpu.VMEM((BT,), jnp.int32),
            pltpu.VMEM((BT,), jnp.int32),
            pltpu.VMEM((C,), jnp.int32),
            pltpu.VMEM((C,), jnp.int32),
            pltpu.SemaphoreType.DMA,
            pltpu.SemaphoreType.DMA,
            pltpu.SemaphoreType.DMA,
            pltpu.SemaphoreType.DMA,
        ],
    )
    return f(src, dst, hw_flat, s2_flat)


def _ln(x, g, b, eps=1e-5):
    m = jnp.mean(x, axis=-1, keepdims=True)
    v = jnp.mean((x - m) ** 2, axis=-1, keepdims=True)
    return (x - m) / jnp.sqrt(v + eps) * g + b


def _pre_part(hsc, hv, w, asrc, adst, hw2, s2, exs):
    hw = jnp.dot(hv, w[...], preferred_element_type=jnp.float32)
    ss = jnp.dot(hw, asrc[...], preferred_element_type=jnp.float32)
    sd = jnp.dot(hw, adst[...], preferred_element_type=jnp.float32)
    hw2[0] = hw[:, 0:128]
    hw2[1] = hw[:, 128:256]
    if hsc * 2 == ss.shape[1]:
        s2[0, :, 0:hsc] = ss[:, 0:hsc]
        s2[0, :, hsc:2 * hsc] = sd[:, 0:hsc]
        s2[1, :, 0:hsc] = ss[:, hsc:2 * hsc]
        s2[1, :, hsc:2 * hsc] = sd[:, hsc:2 * hsc]
    else:
        s2[0, :, 0:1] = ss
        s2[0, :, 1:2] = sd
        s2[1, :, 0:1] = ss
        s2[1, :, 1:2] = sd
    a = ss + sd
    a = jnp.where(a > 0, a, 0.2 * a)
    exs[...] = jnp.exp(a)


def _post_part(hsc, acc, den, exs, hw2, h, rep, bias, g, b):
    num = jnp.concatenate([acc[0], acc[1]], axis=1)
    hwc = jnp.concatenate([hw2[0], hw2[1]], axis=1)
    e = exs[...]
    num = num + jnp.dot(e, rep[...],
                        preferred_element_type=jnp.float32) * hwc
    if hsc * 2 == e.shape[1]:
        denh = jnp.concatenate([den[0][:, 0:hsc], den[1][:, 0:hsc]], axis=1)
    else:
        denh = den[0][:, 0:1]
    d = jnp.dot(denh + e, rep[...], preferred_element_type=jnp.float32)
    xn = num / (d + 1e-16) + bias[...]
    xn = jnp.where(xn > 0, xn, jnp.exp(xn) - 1.0)
    return _ln(xn + h[...], g[...], b[...])


BN = 1000


def _k_front_body(x, emb, w1, b1, w2, b2, g, b, w0, a0s, a0d,
                  h_out, hw2, s2, exs):
    xr = x[...]
    ohc = (xr[:, 0:1].astype(jnp.int32) == lax.broadcasted_iota(
        jnp.int32, (1, 128), 1)).astype(jnp.float32)
    h1 = jnp.maximum(
        jnp.dot(xr.astype(jnp.bfloat16), w1[...],
                preferred_element_type=jnp.float32) + b1[...], 0.0)
    v = jnp.dot(h1, w2[...], preferred_element_type=jnp.float32) + b2[...]
    ce = jnp.dot(ohc, emb[...], preferred_element_type=jnp.float32)
    hv = jnp.concatenate([ce, _ln(v, g[...], b[...])], axis=1)
    h_out[...] = hv
    _pre_part(2, hv, w0, a0s, a0d, hw2, s2, exs)


def _k_front(x, emb, w1p, b1, w2, b2, g, b, w0, a0s, a0d):
    grid = (N // BN,)
    return pl.pallas_call(
        _k_front_body,
        grid=grid,
        in_specs=[
            pl.BlockSpec((BN, 2049), lambda i: (i, 0)),
            pl.BlockSpec((128, 128), lambda i: (0, 0)),
            pl.BlockSpec((2049, 512), lambda i: (0, 0)),
            pl.BlockSpec((1, 512), lambda i: (0, 0)),
            pl.BlockSpec((512, 128), lambda i: (0, 0)),
            pl.BlockSpec((1, 128), lambda i: (0, 0)),
            pl.BlockSpec((1, 128), lambda i: (0, 0)),
            pl.BlockSpec((1, 128), lambda i: (0, 0)),
            pl.BlockSpec((256, 256), lambda i: (0, 0)),
            pl.BlockSpec((256, 4), lambda i: (0, 0)),
            pl.BlockSpec((256, 4), lambda i: (0, 0)),
        ],
        out_specs=[
            pl.BlockSpec((BN, 256), lambda i: (i, 0)),
            pl.BlockSpec((2, BN, 128), lambda i: (0, i, 0)),
            pl.BlockSpec((2, BN, 4), lambda i: (0, i, 0)),
            pl.BlockSpec((BN, 4), lambda i: (i, 0)),
        ],
        out_shape=[
            jax.ShapeDtypeStruct((N, 256), jnp.float32),
            jax.ShapeDtypeStruct((2, N, 128), jnp.float32),
            jax.ShapeDtypeStruct((2, N, 4), jnp.float32),
            jax.ShapeDtypeStruct((N, 4), jnp.float32),
        ],
    )(x, emb, w1p, b1, w2, b2, g, b, w0, a0s, a0d)


def _k_mid_body(acc, den, exs0, hw20, h, rep, bias, g, b, w1, a1s, a1d,
                h1_out, hw2, s2, exs):
    h1v = _post_part(2, acc, den, exs0, hw20, h, rep, bias, g, b)
    h1_out[...] = h1v
    _pre_part(1, h1v, w1, a1s, a1d, hw2, s2, exs)


def _k_mid(acc, den, exs0, hw20, h, rep, bias, g, b, w1, a1s, a1d):
    grid = (N // BN,)
    return pl.pallas_call(
        _k_mid_body,
        grid=grid,
        in_specs=[
            pl.BlockSpec((2, BN, 128), lambda i: (0, i, 0)),
            pl.BlockSpec((2, BN, DEN_W), lambda i: (0, i, 0)),
            pl.BlockSpec((BN, 4), lambda i: (i, 0)),
            pl.BlockSpec((2, BN, 128), lambda i: (0, i, 0)),
            pl.BlockSpec((BN, 256), lambda i: (i, 0)),
            pl.BlockSpec((4, 256), lambda i: (0, 0)),
            pl.BlockSpec((1, 256), lambda i: (0, 0)),
            pl.BlockSpec((1, 256), lambda i: (0, 0)),
            pl.BlockSpec((1, 256), lambda i: (0, 0)),
            pl.BlockSpec((256, 256), lambda i: (0, 0)),
            pl.BlockSpec((256, 1), lambda i: (0, 0)),
            pl.BlockSpec((256, 1), lambda i: (0, 0)),
        ],
        out_specs=[
            pl.BlockSpec((BN, 256), lambda i: (i, 0)),
            pl.BlockSpec((2, BN, 128), lambda i: (0, i, 0)),
            pl.BlockSpec((2, BN, 2), lambda i: (0, i, 0)),
            pl.BlockSpec((BN, 1), lambda i: (i, 0)),
        ],
        out_shape=[
            jax.ShapeDtypeStruct((N, 256), jnp.float32),
            jax.ShapeDtypeStruct((2, N, 128), jnp.float32),
            jax.ShapeDtypeStruct((2, N, 2), jnp.float32),
            jax.ShapeDtypeStruct((N, 1), jnp.float32),
        ],
    )(acc, den, exs0, hw20, h, rep, bias, g, b, w1, a1s, a1d)


def _k_end_body(nsteps, acc, den, exs1, hw21, h1, rep, bias, g, b, bf,
                row, rob, out, psum, cnt):
    i = pl.program_id(0)

    @pl.when(i == 0)
    def _init():
        psum[...] = jnp.zeros_like(psum)
        cnt[...] = jnp.zeros_like(cnt)

    h2v = _post_part(1, acc, den, exs1, hw21, h1, rep, bias, g, b)
    ohb = (bf[...].astype(jnp.int32) == lax.broadcasted_iota(
        jnp.int32, (1, G), 1)).astype(jnp.float32)
    psum[...] += lax.dot_general(ohb, h2v, (((0,), (0,)), ((), ())),
                                 preferred_element_type=jnp.float32)
    cnt[...] += jnp.sum(ohb, axis=0, keepdims=True)

    @pl.when(i == nsteps - 1)
    def _fin():
        pooled = psum[...] / jnp.maximum(cnt[...], 1.0).reshape(G, 1)
        logit = jnp.dot(pooled, row[...],
                        preferred_element_type=jnp.float32) + rob[...]
        out[...] = 1.0 / (1.0 + jnp.exp(-logit))


def _k_end(acc, den, exs1, hw21, h1, rep, bias, g, b, bf, row, rob):
    nsteps = N // BN
    return pl.pallas_call(
        functools.partial(_k_end_body, nsteps),
        grid=(nsteps,),
        in_specs=[
            pl.BlockSpec((2, BN, 128), lambda i: (0, i, 0)),
            pl.BlockSpec((2, BN, DEN_W), lambda i: (0, i, 0)),
            pl.BlockSpec((BN, 1), lambda i: (i, 0)),
            pl.BlockSpec((2, BN, 128), lambda i: (0, i, 0)),
            pl.BlockSpec((BN, 256), lambda i: (i, 0)),
            pl.BlockSpec((1, 256), lambda i: (0, 0)),
            pl.BlockSpec((1, 256), lambda i: (0, 0)),
            pl.BlockSpec((1, 256), lambda i: (0, 0)),
            pl.BlockSpec((1, 256), lambda i: (0, 0)),
            pl.BlockSpec((BN, 1), lambda i: (i, 0)),
            pl.BlockSpec((256, 1), lambda i: (0, 0)),
            pl.BlockSpec((1, 1), lambda i: (0, 0)),
        ],
        out_specs=pl.BlockSpec((G, 1), lambda i: (0, 0)),
        out_shape=jax.ShapeDtypeStruct((G, 1), jnp.float32),
        scratch_shapes=[
            pltpu.VMEM((G, 256), jnp.float32),
            pltpu.VMEM((1, G), jnp.float32),
        ],
    )(acc, den, exs1, hw21, h1, rep, bias, g, b, bf, row, rob)


def _expander(a, heads, oc):
    rows = jnp.repeat(jnp.arange(heads), oc)
    mask = (rows[:, None] == jnp.arange(heads)[None, :]).astype(jnp.float32)
    return a.reshape(heads * oc, 1) * mask


def _rep(heads, colw):
    cols = jnp.arange(256) // colw
    return (jnp.arange(heads)[:, None] == cols[None, :]).astype(jnp.float32)


def kernel(x, edge_index, batch, embed, vp_w1, vp_b1, vp_w2, vp_b2, vp_ln_g,
           vp_ln_b, w0, a_src0, a_dst0, bias0, n0_g, n0_b, w1, a_src1,
           a_dst1, bias1, n1_g, n1_b, ro_w, ro_b):
    emb = jnp.pad(embed, ((0, 128 - embed.shape[0]), (0, 0)))
    w1p = jnp.concatenate(
        [jnp.zeros((1, 512), jnp.bfloat16), vp_w1.astype(jnp.bfloat16)],
        axis=0)
    src = edge_index[0]
    dst = edge_index[1]
    bf = batch.astype(jnp.float32).reshape(N, 1)

    h, hw20, s20, exs0 = _k_front(
        x, emb, w1p, vp_b1.reshape(1, 512), vp_w2, vp_b2.reshape(1, 128),
        vp_ln_g.reshape(1, 128), vp_ln_b.reshape(1, 128), w0,
        _expander(a_src0, 4, 64), _expander(a_dst0, 4, 64))
    acc0, den0 = _edge_agg_call(2, 64, src, dst,
                                hw20.reshape(2 * N, 128),
                                s20.reshape(2 * N, 4))

    h1, hw21, s21, exs1 = _k_mid(
        acc0.reshape(2, N, 128), den0.reshape(2, N, DEN_W), exs0, hw20, h,
        _rep(4, 64), bias0.reshape(1, 256), n0_g.reshape(1, 256),
        n0_b.reshape(1, 256), w1, _expander(a_src1, 1, 256),
        _expander(a_dst1, 1, 256))
    acc1, den1 = _edge_agg_call(1, 128, src, dst,
                                hw21.reshape(2 * N, 128),
                                s21.reshape(2 * N, 2))

    score = _k_end(
        acc1.reshape(2, N, 128), den1.reshape(2, N, DEN_W), exs1, hw21, h1,
        _rep(1, 256), bias1.reshape(1, 256), n1_g.reshape(1, 256),
        n1_b.reshape(1, 256), bf, ro_w, ro_b.reshape(1, 1))
    return score.reshape(G)


# speedup vs baseline: 40.2475x; 1.0866x over previous
import functools

import jax
import jax.numpy as jnp
from jax import lax
from jax.experimental import pallas as pl
from jax.experimental.pallas import tpu as pltpu
from jax.experimental.pallas import tpu_sc as plsc

N = 10000
E = 160000
G = 64
HID = 256

NTILE = 16
EPT = E // NTILE
C = 80
BT = 2000
NCHUNK = EPT // C
WB_TILES = 10
WB_ROWS = N // WB_TILES
WB_CH = 40
DEN_W = 16


CE = 2000


def _edge_agg_call(hsc, colw, src, dst, hw_flat, s2_flat):
    mesh = plsc.VectorSubcoreMesh(core_axis_name="c", subcore_axis_name="s")
    NCB = BT // C
    NB = EPT // BT

    def body(src_hbm, dst_hbm, hw_hbm, s2_hbm, acc_hbm, den_hbm,
             acc_sh, den_sh, rows0, rows1, exb0, exb1, sbs0, sbs1,
             sbd0, sbd1, srcb, dstb, dstob, dstc0, dstc1,
             semg0, semg1, sems0, sems1):
        c = lax.axis_index("c")
        t = lax.axis_index("s")
        cN = c * N
        base = t * WB_ROWS
        iota16 = jnp.arange(16, dtype=jnp.int32)
        zero16 = jnp.zeros((16,), jnp.float32)
        rows = (rows0, rows1)
        exb = (exb0, exb1)
        sbs = (sbs0, sbs1)
        sbd = (sbd0, sbd1)
        dstc = (dstc0, dstc1)
        semg = (semg0, semg1)
        sems = (sems0, sems1)
        ebase = t * EPT

        def zrow(r, _):
            for v in range(128 // 16):
                rows0[r, pl.ds(v * 16, 16)] = zero16
            exb0[r, pl.ds(0, 16)] = zero16
            exb1[r, pl.ds(0, 16)] = zero16
            return 0
        lax.fori_loop(0, C, zrow, 0)

        @pl.when(t < WB_TILES)
        def _zero():
            for j in range(WB_ROWS // WB_CH):
                pltpu.sync_copy(rows0.at[pl.ds(0, WB_CH), :],
                                acc_sh.at[pl.ds(base + j * WB_CH, WB_CH), :])
                pltpu.sync_copy(exb0.at[pl.ds(0, WB_CH), :],
                                den_sh.at[pl.ds(base + j * WB_CH, WB_CH), :])
        plsc.subcore_barrier()

        def start_chunk(j, s):
            pltpu.async_copy(hw_hbm.at[srcb.at[pl.ds(j * C, C)]],
                             rows[s], semg[s])
            pltpu.async_copy(s2_hbm.at[srcb.at[pl.ds(j * C, C)]],
                             sbs[s], semg[s])
            pltpu.async_copy(s2_hbm.at[dstob.at[pl.ds(j * C, C)]],
                             sbd[s], semg[s])

        def do_chunk(j, s):
            o = 1 - s
            pltpu.make_async_copy(hw_hbm.at[srcb.at[pl.ds(0, C)]],
                                  rows[s], semg[s]).wait()
            pltpu.make_async_copy(s2_hbm.at[srcb.at[pl.ds(0, C)]],
                                  sbs[s], semg[s]).wait()
            pltpu.make_async_copy(s2_hbm.at[dstob.at[pl.ds(0, C)]],
                                  sbd[s], semg[s]).wait()

            @pl.when(j >= 1)
            def _drain_other():
                pltpu.make_async_copy(rows[o], acc_sh.at[dstc[o]],
                                      sems[o]).wait()
                pltpu.make_async_copy(exb[o], den_sh.at[dstc[o]],
                                      sems[o]).wait()

            @pl.when(j < NCB - 1)
            def _prefetch():
                start_chunk(j + 1, o)

            for k in range(C // 16):
                dstc[s][pl.ds(k * 16, 16)] = dstb[pl.ds(j * C + k * 16, 16)]

            @functools.partial(plsc.parallel_loop, 0, C // 16, unroll=2)
            def group(g):
                ev = g * 16 + iota16
                for h in range(hsc):
                    hcol = jnp.full((16,), h, jnp.int32)
                    a = (plsc.load_gather(sbs[s], [ev, hcol])
                         + plsc.load_gather(sbd[s], [ev, hcol + hsc]))
                    a = jnp.where(a > 0, a, 0.2 * a)
                    plsc.store_scatter(exb[s], [ev, hcol], jnp.exp(a))

            @functools.partial(plsc.parallel_loop, 0, C, unroll=4)
            def edge(e):
                exrow = exb[s][e, pl.ds(0, 16)]
                for h in range(hsc):
                    bc = jnp.full((16,), exrow[h], jnp.float32)
                    for v in range((h * colw) // 16, ((h + 1) * colw) // 16):
                        rows[s][e, pl.ds(v * 16, 16)] = (
                            rows[s][e, pl.ds(v * 16, 16)] * bc)

            pltpu.async_copy(rows[s], acc_sh.at[dstc[s]], sems[s], add=True)
            pltpu.async_copy(exb[s], den_sh.at[dstc[s]], sems[s], add=True)

        for b in range(NB):
            pltpu.sync_copy(src_hbm.at[pl.ds(ebase + b * BT, BT)], srcb)
            pltpu.sync_copy(dst_hbm.at[pl.ds(ebase + b * BT, BT)], dstb)

            def adj(k, _):
                srcb[pl.ds(k * 16, 16)] = srcb[pl.ds(k * 16, 16)] + cN
                dstob[pl.ds(k * 16, 16)] = dstb[pl.ds(k * 16, 16)] + cN
                return 0
            lax.fori_loop(0, BT // 16, adj, 0)

            start_chunk(0, 0)

            def inner(j, _):
                @pl.when(j % 2 == 0)
                def _even():
                    do_chunk(j, 0)

                @pl.when(j % 2 == 1)
                def _odd():
                    do_chunk(j, 1)
                return 0
            lax.fori_loop(0, NCB, inner, 0)

            s_last = (NCB - 1) % 2
            pltpu.make_async_copy(rows[s_last], acc_sh.at[dstc[s_last]],
                                  sems[s_last]).wait()
            pltpu.make_async_copy(exb[s_last], den_sh.at[dstc[s_last]],
                                  sems[s_last]).wait()

        plsc.subcore_barrier()

        @pl.when(t < WB_TILES)
        def _writeback():
            for j in range(WB_ROWS // WB_CH):
                o = base + j * WB_CH
                pltpu.sync_copy(acc_sh.at[pl.ds(o, WB_CH), :],
                                acc_hbm.at[pl.ds(cN + o, WB_CH), :])
                pltpu.sync_copy(den_sh.at[pl.ds(o, WB_CH), :],
                                den_hbm.at[pl.ds(cN + o, WB_CH), :])

    f = pl.kernel(
        body,
        out_type=(jax.ShapeDtypeStruct((2 * N, 128), jnp.float32),
                  jax.ShapeDtypeStruct((2 * N, DEN_W), jnp.float32)),
        mesh=mesh,
        compiler_params=pltpu.CompilerParams(needs_layout_passes=False, use_tc_tiling_on_sc=False),
        scratch_types=[
            pltpu.VMEM_SHARED((N, 128), jnp.float32),
            pltpu.VMEM_SHARED((N, DEN_W), jnp.float32),
            pltpu.VMEM((C, 128), jnp.float32),
            pltpu.VMEM((C, 128), jnp.float32),
            pltpu.VMEM((C, DEN_W), jnp.float32),
            pltpu.VMEM((C, DEN_W), jnp.float32),
            pltpu.VMEM((C, 2 * hsc), jnp.float32),
            pltpu.VMEM((C, 2 * hsc), jnp.float32),
            pltpu.VMEM((C, 2 * hsc), jnp.float32),
            pltpu.VMEM((C, 2 * hsc), jnp.float32),
            pltpu.VMEM((BT,), jnp.int32),
            pltpu.VMEM((BT,), jnp.int32),
            pltpu.VMEM((BT,), jnp.int32),
            pltpu.VMEM((C,), jnp.int32),
            pltpu.VMEM((C,), jnp.int32),
            pltpu.SemaphoreType.DMA,
            pltpu.SemaphoreType.DMA,
            pltpu.SemaphoreType.DMA,
            pltpu.SemaphoreType.DMA,
        ],
    )
    return f(src, dst, hw_flat, s2_flat)


def _ln(x, g, b, eps=1e-5):
    m = jnp.mean(x, axis=-1, keepdims=True)
    v = jnp.mean((x - m) ** 2, axis=-1, keepdims=True)
    return (x - m) / jnp.sqrt(v + eps) * g + b


def _pre_part(hsc, hv, w, asrc, adst, hw2, s2, exs):
    hw = jnp.dot(hv, w[...], preferred_element_type=jnp.float32)
    ss = jnp.dot(hw, asrc[...], preferred_element_type=jnp.float32)
    sd = jnp.dot(hw, adst[...], preferred_element_type=jnp.float32)
    hw2[0] = hw[:, 0:128]
    hw2[1] = hw[:, 128:256]
    if hsc * 2 == ss.shape[1]:
        s2[0, :, 0:hsc] = ss[:, 0:hsc]
        s2[0, :, hsc:2 * hsc] = sd[:, 0:hsc]
        s2[1, :, 0:hsc] = ss[:, hsc:2 * hsc]
        s2[1, :, hsc:2 * hsc] = sd[:, hsc:2 * hsc]
    else:
        s2[0, :, 0:1] = ss
        s2[0, :, 1:2] = sd
        s2[1, :, 0:1] = ss
        s2[1, :, 1:2] = sd
    a = ss + sd
    a = jnp.where(a > 0, a, 0.2 * a)
    exs[...] = jnp.exp(a)


def _post_part(hsc, acc, den, exs, hw2, h, rep, bias, g, b):
    num = jnp.concatenate([acc[0], acc[1]], axis=1)
    hwc = jnp.concatenate([hw2[0], hw2[1]], axis=1)
    e = exs[...]
    num = num + jnp.dot(e, rep[...],
                        preferred_element_type=jnp.float32) * hwc
    if hsc * 2 == e.shape[1]:
        denh = jnp.concatenate([den[0][:, 0:hsc], den[1][:, 0:hsc]], axis=1)
    else:
        denh = den[0][:, 0:1]
    d = jnp.dot(denh + e, rep[...], preferred_element_type=jnp.float32)
    xn = num / (d + 1e-16) + bias[...]
    xn = jnp.where(xn > 0, xn, jnp.exp(xn) - 1.0)
    return _ln(xn + h[...], g[...], b[...])


BN = 1000


def _k_front_body(x, emb, w1, b1, w2, b2, g, b, w0, a0s, a0d,
                  h_out, hw2, s2, exs):
    xr = x[...]
    ohc = (xr[:, 0:1].astype(jnp.int32) == lax.broadcasted_iota(
        jnp.int32, (1, 128), 1)).astype(jnp.float32)
    h1 = jnp.maximum(
        jnp.dot(xr.astype(jnp.bfloat16), w1[...],
                preferred_element_type=jnp.float32) + b1[...], 0.0)
    v = jnp.dot(h1, w2[...], preferred_element_type=jnp.float32) + b2[...]
    ce = jnp.dot(ohc, emb[...], preferred_element_type=jnp.float32)
    hv = jnp.concatenate([ce, _ln(v, g[...], b[...])], axis=1)
    h_out[...] = hv
    _pre_part(2, hv, w0, a0s, a0d, hw2, s2, exs)


def _k_front(x, emb, w1p, b1, w2, b2, g, b, w0, a0s, a0d):
    grid = (N // BN,)
    return pl.pallas_call(
        _k_front_body,
        grid=grid,
        in_specs=[
            pl.BlockSpec((BN, 2049), lambda i: (i, 0)),
            pl.BlockSpec((128, 128), lambda i: (0, 0)),
            pl.BlockSpec((2049, 512), lambda i: (0, 0)),
            pl.BlockSpec((1, 512), lambda i: (0, 0)),
            pl.BlockSpec((512, 128), lambda i: (0, 0)),
            pl.BlockSpec((1, 128), lambda i: (0, 0)),
            pl.BlockSpec((1, 128), lambda i: (0, 0)),
            pl.BlockSpec((1, 128), lambda i: (0, 0)),
            pl.BlockSpec((256, 256), lambda i: (0, 0)),
            pl.BlockSpec((256, 4), lambda i: (0, 0)),
            pl.BlockSpec((256, 4), lambda i: (0, 0)),
        ],
        out_specs=[
            pl.BlockSpec((BN, 256), lambda i: (i, 0)),
            pl.BlockSpec((2, BN, 128), lambda i: (0, i, 0)),
            pl.BlockSpec((2, BN, 4), lambda i: (0, i, 0)),
            pl.BlockSpec((BN, 4), lambda i: (i, 0)),
        ],
        out_shape=[
            jax.ShapeDtypeStruct((N, 256), jnp.float32),
            jax.ShapeDtypeStruct((2, N, 128), jnp.float32),
            jax.ShapeDtypeStruct((2, N, 4), jnp.float32),
            jax.ShapeDtypeStruct((N, 4), jnp.float32),
        ],
    )(x, emb, w1p, b1, w2, b2, g, b, w0, a0s, a0d)


def _k_mid_body(acc, den, exs0, hw20, h, rep, bias, g, b, w1, a1s, a1d,
                h1_out, hw2, s2, exs):
    h1v = _post_part(2, acc, den, exs0, hw20, h, rep, bias, g, b)
    h1_out[...] = h1v
    _pre_part(1, h1v, w1, a1s, a1d, hw2, s2, exs)


def _k_mid(acc, den, exs0, hw20, h, rep, bias, g, b, w1, a1s, a1d):
    grid = (N // BN,)
    return pl.pallas_call(
        _k_mid_body,
        grid=grid,
        in_specs=[
            pl.BlockSpec((2, BN, 128), lambda i: (0, i, 0)),
            pl.BlockSpec((2, BN, DEN_W), lambda i: (0, i, 0)),
            pl.BlockSpec((BN, 4), lambda i: (i, 0)),
            pl.BlockSpec((2, BN, 128), lambda i: (0, i, 0)),
            pl.BlockSpec((BN, 256), lambda i: (i, 0)),
            pl.BlockSpec((4, 256), lambda i: (0, 0)),
            pl.BlockSpec((1, 256), lambda i: (0, 0)),
            pl.BlockSpec((1, 256), lambda i: (0, 0)),
            pl.BlockSpec((1, 256), lambda i: (0, 0)),
            pl.BlockSpec((256, 256), lambda i: (0, 0)),
            pl.BlockSpec((256, 1), lambda i: (0, 0)),
            pl.BlockSpec((256, 1), lambda i: (0, 0)),
        ],
        out_specs=[
            pl.BlockSpec((BN, 256), lambda i: (i, 0)),
            pl.BlockSpec((2, BN, 128), lambda i: (0, i, 0)),
            pl.BlockSpec((2, BN, 2), lambda i: (0, i, 0)),
            pl.BlockSpec((BN, 1), lambda i: (i, 0)),
        ],
        out_shape=[
            jax.ShapeDtypeStruct((N, 256), jnp.float32),
            jax.ShapeDtypeStruct((2, N, 128), jnp.float32),
            jax.ShapeDtypeStruct((2, N, 2), jnp.float32),
            jax.ShapeDtypeStruct((N, 1), jnp.float32),
        ],
    )(acc, den, exs0, hw20, h, rep, bias, g, b, w1, a1s, a1d)


def _k_end_body(nsteps, acc, den, exs1, hw21, h1, rep, bias, g, b, bf,
                row, rob, out, psum, cnt):
    i = pl.program_id(0)

    @pl.when(i == 0)
    def _init():
        psum[...] = jnp.zeros_like(psum)
        cnt[...] = jnp.zeros_like(cnt)

    h2v = _post_part(1, acc, den, exs1, hw21, h1, rep, bias, g, b)
    ohb = (bf[...].astype(jnp.int32) == lax.broadcasted_iota(
        jnp.int32, (1, G), 1)).astype(jnp.float32)
    psum[...] += lax.dot_general(ohb, h2v, (((0,), (0,)), ((), ())),
                                 preferred_element_type=jnp.float32)
    cnt[...] += jnp.sum(ohb, axis=0, keepdims=True)

    @pl.when(i == nsteps - 1)
    def _fin():
        pooled = psum[...] / jnp.maximum(cnt[...], 1.0).reshape(G, 1)
        logit = jnp.dot(pooled, row[...],
                        preferred_element_type=jnp.float32) + rob[...]
        out[...] = 1.0 / (1.0 + jnp.exp(-logit))


def _k_end(acc, den, exs1, hw21, h1, rep, bias, g, b, bf, row, rob):
    nsteps = N // BN
    return pl.pallas_call(
        functools.partial(_k_end_body, nsteps),
        grid=(nsteps,),
        in_specs=[
            pl.BlockSpec((2, BN, 128), lambda i: (0, i, 0)),
            pl.BlockSpec((2, BN, DEN_W), lambda i: (0, i, 0)),
            pl.BlockSpec((BN, 1), lambda i: (i, 0)),
            pl.BlockSpec((2, BN, 128), lambda i: (0, i, 0)),
            pl.BlockSpec((BN, 256), lambda i: (i, 0)),
            pl.BlockSpec((1, 256), lambda i: (0, 0)),
            pl.BlockSpec((1, 256), lambda i: (0, 0)),
            pl.BlockSpec((1, 256), lambda i: (0, 0)),
            pl.BlockSpec((1, 256), lambda i: (0, 0)),
            pl.BlockSpec((BN, 1), lambda i: (i, 0)),
            pl.BlockSpec((256, 1), lambda i: (0, 0)),
            pl.BlockSpec((1, 1), lambda i: (0, 0)),
        ],
        out_specs=pl.BlockSpec((G, 1), lambda i: (0, 0)),
        out_shape=jax.ShapeDtypeStruct((G, 1), jnp.float32),
        scratch_shapes=[
            pltpu.VMEM((G, 256), jnp.float32),
            pltpu.VMEM((1, G), jnp.float32),
        ],
    )(acc, den, exs1, hw21, h1, rep, bias, g, b, bf, row, rob)


def _expander(a, heads, oc):
    rows = jnp.repeat(jnp.arange(heads), oc)
    mask = (rows[:, None] == jnp.arange(heads)[None, :]).astype(jnp.float32)
    return a.reshape(heads * oc, 1) * mask


def _rep(heads, colw):
    cols = jnp.arange(256) // colw
    return (jnp.arange(heads)[:, None] == cols[None, :]).astype(jnp.float32)


def kernel(x, edge_index, batch, embed, vp_w1, vp_b1, vp_w2, vp_b2, vp_ln_g,
           vp_ln_b, w0, a_src0, a_dst0, bias0, n0_g, n0_b, w1, a_src1,
           a_dst1, bias1, n1_g, n1_b, ro_w, ro_b):
    emb = jnp.pad(embed, ((0, 128 - embed.shape[0]), (0, 0)))
    w1p = jnp.concatenate(
        [jnp.zeros((1, 512), jnp.bfloat16), vp_w1.astype(jnp.bfloat16)],
        axis=0)
    src = edge_index[0]
    dst = edge_index[1]
    bf = batch.astype(jnp.float32).reshape(N, 1)

    h, hw20, s20, exs0 = _k_front(
        x, emb, w1p, vp_b1.reshape(1, 512), vp_w2, vp_b2.reshape(1, 128),
        vp_ln_g.reshape(1, 128), vp_ln_b.reshape(1, 128), w0,
        _expander(a_src0, 4, 64), _expander(a_dst0, 4, 64))
    acc0, den0 = _edge_agg_call(2, 64, src, dst,
                                hw20.reshape(2 * N, 128),
                                s20.reshape(2 * N, 4))

    h1, hw21, s21, exs1 = _k_mid(
        acc0.reshape(2, N, 128), den0.reshape(2, N, DEN_W), exs0, hw20, h,
        _rep(4, 64), bias0.reshape(1, 256), n0_g.reshape(1, 256),
        n0_b.reshape(1, 256), w1, _expander(a_src1, 1, 256),
        _expander(a_dst1, 1, 256))
    acc1, den1 = _edge_agg_call(1, 128, src, dst,
                                hw21.reshape(2 * N, 128),
                                s21.reshape(2 * N, 2))

    score = _k_end(
        acc1.reshape(2, N, 128), den1.reshape(2, N, DEN_W), exs1, hw21, h1,
        _rep(1, 256), bias1.reshape(1, 256), n1_g.reshape(1, 256),
        n1_b.reshape(1, 256), bf, ro_w, ro_b.reshape(1, 1))
    return score.reshape(G)
